# Initial kernel scaffold; baseline (speedup 1.0000x reference)
import jax
import jax.numpy as jnp
from jax.experimental import pallas as pl


def kernel(x, qkv_w, qkv_b, w1_w, w2_w, mlp1_w, mlp1_b, mlp2_w, mlp2_b, gate_w, gate_b, out_w, out_b):
    raise NotImplementedError("write your pallas kernel here")


# speedup vs baseline: 3.6477x; 3.6477x over previous
import functools
from itertools import combinations

import numpy as np
import jax
import jax.numpy as jnp
from jax.experimental import pallas as pl
from jax.experimental.pallas import tpu as pltpu

_D = 1024
_H = 16
_DH = 64
_P = 4
_PD = 6
_NG = 21
_NGP = 24
_DECAY = 0.99
_C = 256
_KC = 256
_RC = 512
_SCALE = _DH ** -0.5
_LN_DECAY = float(np.log(_DECAY))
_DECAY_C = float(_DECAY ** _C)

_PAIRS = list(combinations(range(_P), 2))
_TI, _TJ = np.triu_indices(_PD)


def _np_consts():
    ea = np.zeros((8, 8), np.float32)
    eb = np.zeros((8, 8), np.float32)
    ec = np.zeros((8, 8), np.float32)
    ed = np.zeros((8, 8), np.float32)
    for kk, (i, j) in enumerate(_PAIRS):
        ea[i, kk] = 1.0
        eb[4 + j, kk] = 1.0
        ec[j, kk] = 1.0
        ed[4 + i, kk] = 1.0
    eti = np.zeros((8, _NGP), np.float32)
    etj = np.zeros((8, _NGP), np.float32)
    for kk in range(_NG):
        eti[_TI[kk], kk] = 1.0
        etj[_TJ[kk], kk] = 1.0
    i = np.arange(_C)[:, None]
    s = np.arange(_C)[None, :]
    lmat = np.where(s < i, _DECAY ** np.maximum(i - 1 - s, 0), 0.0).astype(np.float32)
    lfull = np.zeros((_C + 8, _C), np.float32)
    lfull[:_C] = lmat
    lfull[_C, :] = _DECAY ** (_C - 1 - np.arange(_C))
    return ea, eb, ec, ed, eti, etj, lfull


_EA, _EB, _EC, _ED, _ETI, _ETJ, _LFULL = _np_consts()


def _proj_kernel(x_ref, qw_ref, qb_ref, w1_ref, w2_ref, gw_ref, gb_ref,
                 qkv_ref, p1_ref, p2_ref, gl_ref):
    xb = x_ref[...]
    qkv_ref[...] = jnp.dot(xb, qw_ref[...], preferred_element_type=jnp.float32) + qb_ref[...]
    p1_ref[...] = jnp.dot(xb, w1_ref[...], preferred_element_type=jnp.float32)
    p2_ref[...] = jnp.dot(xb, w2_ref[...], preferred_element_type=jnp.float32)
    gl_ref[...] = jnp.dot(xb, gw_ref[...], preferred_element_type=jnp.float32) + gb_ref[...]


def _attn_gram_kernel(q_ref, k_ref, v_ref, pw_ref, gl_ref, l_ref,
                      ea_ref, eb_ref, ec_ref, ed_ref, eti_ref, etj_ref,
                      m1_ref, m1b_ref, m2_ref, m2b_ref,
                      out_ref, s_ref):
    bh = pl.program_id(0)
    qc = pl.program_id(1)
    h = bh % _H
    t0 = qc * _C

    @pl.when(qc == 0)
    def _():
        s_ref[...] = jnp.zeros((1, _NGP), jnp.float32)

    q = q_ref[0, 0]

    def body(j, carry):
        m, l, acc = carry
        off = pl.multiple_of(j * _KC, _KC)
        kc = k_ref[0, 0, pl.ds(off, _KC), :]
        sc = jax.lax.dot_general(q, kc, (((1,), (1,)), ((), ())),
                                 preferred_element_type=jnp.float32) * _SCALE
        colg = j * _KC + jax.lax.broadcasted_iota(jnp.int32, (_C, _KC), 1)
        rowg = t0 + jax.lax.broadcasted_iota(jnp.int32, (_C, _KC), 0)
        sc = jnp.where(colg > rowg, -1e30, sc)
        m_new = jnp.maximum(m, jnp.max(sc, axis=1, keepdims=True))
        alpha = jnp.exp(m - m_new)
        p = jnp.exp(sc - m_new)
        l_new = l * alpha + jnp.sum(p, axis=1, keepdims=True)
        vc = v_ref[0, 0, pl.ds(off, _KC), :]
        acc_new = acc * alpha + jnp.dot(p, vc, preferred_element_type=jnp.float32)
        return m_new, l_new, acc_new

    m0 = jnp.full((_C, 1), -1e30, jnp.float32)
    l0 = jnp.zeros((_C, 1), jnp.float32)
    a0 = jnp.zeros((_C, _DH), jnp.float32)
    m, l, acc = jax.lax.fori_loop(0, qc + 1, body, (m0, l0, a0))
    seq = acc / l

    pw = pw_ref[0, 0]
    a = jnp.dot(pw, ea_ref[...], preferred_element_type=jnp.float32)
    b = jnp.dot(pw, eb_ref[...], preferred_element_type=jnp.float32)
    c = jnp.dot(pw, ec_ref[...], preferred_element_type=jnp.float32)
    d = jnp.dot(pw, ed_ref[...], preferred_element_type=jnp.float32)
    parts = a * b - c * d
    s2 = jnp.sum(parts * parts, axis=1, keepdims=True)
    nr = jnp.maximum(jnp.sqrt(s2), 1e-12)
    wl = parts / nr
    u = jnp.dot(wl, eti_ref[...], preferred_element_type=jnp.float32)
    v = jnp.dot(wl, etj_ref[...], preferred_element_type=jnp.float32)
    o = u * v

    carry_s = s_ref[...]
    dp = jnp.exp(jax.lax.broadcasted_iota(jnp.int32, (_C, _NGP), 0).astype(jnp.float32) * _LN_DECAY)
    gf = dp * carry_s + jnp.dot(l_ref[0:_C, :], o, preferred_element_type=jnp.float32)
    s_ref[...] = _DECAY_C * carry_s + jnp.dot(l_ref[_C:_C + 1, :], o,
                                              preferred_element_type=jnp.float32)

    pre = jnp.dot(gf, m1_ref[...], preferred_element_type=jnp.float32) + m1b_ref[...]
    h1 = 0.5 * pre * (1.0 + jax.lax.erf(pre * 0.7071067811865476))
    mem = jnp.dot(h1, m2_ref[...], preferred_element_type=jnp.float32) + m2b_ref[...]

    oh = (jax.lax.broadcasted_iota(jnp.int32, (_C, _H), 1) == h).astype(jnp.float32)
    gcol = jnp.sum(gl_ref[0] * oh, axis=1, keepdims=True)
    gate = jax.nn.sigmoid(gcol)

    out_ref[0, 0] = seq + gate * mem


def _out_kernel(c_ref, w_ref, b_ref, o_ref):
    o_ref[...] = jnp.dot(c_ref[...], w_ref[...], preferred_element_type=jnp.float32) + b_ref[...]


def kernel(x, qkv_w, qkv_b, w1_w, w2_w, mlp1_w, mlp1_b, mlp2_w, mlp2_b,
           gate_w, gate_b, out_w, out_b):
    bsz, t, dm = x.shape
    f32 = jnp.float32
    xf = x.reshape(bsz * t, dm)
    rows = bsz * t
    ngrid = rows // _RC

    qkv, p1, p2, glog = pl.pallas_call(
        _proj_kernel,
        grid=(ngrid,),
        in_specs=[
            pl.BlockSpec((_RC, dm), lambda i: (i, 0)),
            pl.BlockSpec((dm, 3 * dm), lambda i: (0, 0)),
            pl.BlockSpec((1, 3 * dm), lambda i: (0, 0)),
            pl.BlockSpec((dm, _H * _P), lambda i: (0, 0)),
            pl.BlockSpec((dm, _H * _P), lambda i: (0, 0)),
            pl.BlockSpec((dm, _H), lambda i: (0, 0)),
            pl.BlockSpec((1, _H), lambda i: (0, 0)),
        ],
        out_specs=[
            pl.BlockSpec((_RC, 3 * dm), lambda i: (i, 0)),
            pl.BlockSpec((_RC, _H * _P), lambda i: (i, 0)),
            pl.BlockSpec((_RC, _H * _P), lambda i: (i, 0)),
            pl.BlockSpec((_RC, _H), lambda i: (i, 0)),
        ],
        out_shape=[
            jax.ShapeDtypeStruct((rows, 3 * dm), f32),
            jax.ShapeDtypeStruct((rows, _H * _P), f32),
            jax.ShapeDtypeStruct((rows, _H * _P), f32),
            jax.ShapeDtypeStruct((rows, _H), f32),
        ],
        compiler_params=pltpu.CompilerParams(
            dimension_semantics=("parallel",),
        ),
    )(xf, qkv_w, qkv_b.reshape(1, -1), w1_w, w2_w, gate_w, gate_b.reshape(1, -1))

    qkvh = jnp.transpose(qkv.reshape(bsz, t, 3 * _H, _DH), (0, 2, 1, 3))
    glog3 = glog.reshape(bsz, t, _H)
    p1r = p1.reshape(bsz, t, _H, _P)
    p1s = jnp.concatenate([jnp.zeros((bsz, 1, _H, _P), f32), p1r[:, :-1]], axis=1)
    p2r = p2.reshape(bsz, t, _H, _P)
    pw = jnp.concatenate([p1s, p2r], axis=-1)
    pwt = jnp.transpose(pw, (0, 2, 1, 3))

    m1p = jnp.concatenate([mlp1_w, jnp.zeros((_NGP - _NG, _DH), f32)], axis=0)

    nq = t // _C
    hh = _H

    combined = pl.pallas_call(
        _attn_gram_kernel,
        grid=(bsz * _H, nq),
        in_specs=[
            pl.BlockSpec((1, 1, _C, _DH), lambda bh, qc: (bh // hh, bh % hh, qc, 0)),
            pl.BlockSpec((1, 1, t, _DH), lambda bh, qc: (bh // hh, hh + bh % hh, 0, 0)),
            pl.BlockSpec((1, 1, t, _DH), lambda bh, qc: (bh // hh, 2 * hh + bh % hh, 0, 0)),
            pl.BlockSpec((1, 1, _C, 8), lambda bh, qc: (bh // hh, bh % hh, qc, 0)),
            pl.BlockSpec((1, _C, _H), lambda bh, qc: (bh // hh, qc, 0)),
            pl.BlockSpec((_C + 8, _C), lambda bh, qc: (0, 0)),
            pl.BlockSpec((8, 8), lambda bh, qc: (0, 0)),
            pl.BlockSpec((8, 8), lambda bh, qc: (0, 0)),
            pl.BlockSpec((8, 8), lambda bh, qc: (0, 0)),
            pl.BlockSpec((8, 8), lambda bh, qc: (0, 0)),
            pl.BlockSpec((8, _NGP), lambda bh, qc: (0, 0)),
            pl.BlockSpec((8, _NGP), lambda bh, qc: (0, 0)),
            pl.BlockSpec((_NGP, _DH), lambda bh, qc: (0, 0)),
            pl.BlockSpec((1, _DH), lambda bh, qc: (0, 0)),
            pl.BlockSpec((_DH, _DH), lambda bh, qc: (0, 0)),
            pl.BlockSpec((1, _DH), lambda bh, qc: (0, 0)),
        ],
        out_specs=pl.BlockSpec((1, 1, _C, _DH), lambda bh, qc: (bh // hh, bh % hh, qc, 0)),
        out_shape=jax.ShapeDtypeStruct((bsz, _H, t, _DH), f32),
        scratch_shapes=[pltpu.VMEM((1, _NGP), f32)],
        compiler_params=pltpu.CompilerParams(
            dimension_semantics=("parallel", "arbitrary"),
        ),
    )(qkvh, qkvh, qkvh, pwt, glog3,
      jnp.asarray(_LFULL), jnp.asarray(_EA), jnp.asarray(_EB), jnp.asarray(_EC),
      jnp.asarray(_ED), jnp.asarray(_ETI), jnp.asarray(_ETJ),
      m1p, mlp1_b.reshape(1, -1), mlp2_w, mlp2_b.reshape(1, -1))

    out = pl.pallas_call(
        _out_kernel,
        grid=(ngrid,),
        in_specs=[
            pl.BlockSpec((_RC, dm), lambda i: (i, 0)),
            pl.BlockSpec((dm, dm), lambda i: (0, 0)),
            pl.BlockSpec((1, dm), lambda i: (0, 0)),
        ],
        out_specs=pl.BlockSpec((_RC, dm), lambda i: (i, 0)),
        out_shape=jax.ShapeDtypeStruct((rows, dm), f32),
        compiler_params=pltpu.CompilerParams(
            dimension_semantics=("parallel",),
        ),
    )(jnp.transpose(combined, (0, 2, 1, 3)).reshape(rows, dm),
      out_w, out_b.reshape(1, -1))

    return out.reshape(bsz, t, dm)


# speedup vs baseline: 3.6858x; 1.0104x over previous
import functools
from itertools import combinations

import numpy as np
import jax
import jax.numpy as jnp
from jax.experimental import pallas as pl
from jax.experimental.pallas import tpu as pltpu

_D = 1024
_H = 16
_DH = 64
_P = 4
_PD = 6
_NG = 21
_NGP = 24
_DECAY = 0.99
_C = 256
_KC = 256
_RC = 512
_SCALE = _DH ** -0.5
_LN_DECAY = float(np.log(_DECAY))
_DECAY_C = float(_DECAY ** _C)

_PAIRS = list(combinations(range(_P), 2))
_TI, _TJ = np.triu_indices(_PD)


def _np_consts():
    ea = np.zeros((8, 8), np.float32)
    eb = np.zeros((8, 8), np.float32)
    ec = np.zeros((8, 8), np.float32)
    ed = np.zeros((8, 8), np.float32)
    for kk, (i, j) in enumerate(_PAIRS):
        ea[i, kk] = 1.0
        eb[4 + j, kk] = 1.0
        ec[j, kk] = 1.0
        ed[4 + i, kk] = 1.0
    eti = np.zeros((8, _NGP), np.float32)
    etj = np.zeros((8, _NGP), np.float32)
    for kk in range(_NG):
        eti[_TI[kk], kk] = 1.0
        etj[_TJ[kk], kk] = 1.0
    i = np.arange(_C)[:, None]
    s = np.arange(_C)[None, :]
    lmat = np.where(s < i, _DECAY ** np.maximum(i - 1 - s, 0), 0.0).astype(np.float32)
    lfull = np.zeros((_C + 8, _C), np.float32)
    lfull[:_C] = lmat
    lfull[_C, :] = _DECAY ** (_C - 1 - np.arange(_C))
    return ea, eb, ec, ed, eti, etj, lfull


_EA, _EB, _EC, _ED, _ETI, _ETJ, _LFULL = _np_consts()


def _proj_kernel(x_ref, qw_ref, qb_ref, w1_ref, w2_ref, gw_ref, gb_ref,
                 qkv_ref, p1_ref, p2_ref, gl_ref):
    xb = x_ref[...]
    qkv = jnp.dot(xb, qw_ref[...], preferred_element_type=jnp.float32) + qb_ref[...]
    qkv_ref[...] = qkv.astype(jnp.bfloat16)
    p1_ref[...] = jnp.dot(xb, w1_ref[...], preferred_element_type=jnp.float32)
    p2_ref[...] = jnp.dot(xb, w2_ref[...], preferred_element_type=jnp.float32)
    gl_ref[...] = jnp.dot(xb, gw_ref[...], preferred_element_type=jnp.float32) + gb_ref[...]


def _attn_gram_kernel(q_ref, k_ref, v_ref, pw_ref, gl_ref, l_ref,
                      ea_ref, eb_ref, ec_ref, ed_ref, eti_ref, etj_ref,
                      m1_ref, m1b_ref, m2_ref, m2b_ref,
                      out_ref, s_ref):
    bh = pl.program_id(0)
    qc = pl.program_id(1)
    h = bh % _H
    t0 = qc * _C

    @pl.when(qc == 0)
    def _():
        s_ref[...] = jnp.zeros((1, _NGP), jnp.float32)

    q = q_ref[0, 0]

    def body(j, carry):
        m, l, acc = carry
        off = pl.multiple_of(j * _KC, _KC)
        kc = k_ref[0, 0, pl.ds(off, _KC), :]
        sc = jax.lax.dot_general(q, kc, (((1,), (1,)), ((), ())),
                                 preferred_element_type=jnp.float32) * _SCALE
        colg = j * _KC + jax.lax.broadcasted_iota(jnp.int32, (_C, _KC), 1)
        rowg = t0 + jax.lax.broadcasted_iota(jnp.int32, (_C, _KC), 0)
        sc = jnp.where(colg > rowg, -1e30, sc)
        m_new = jnp.maximum(m, jnp.max(sc, axis=1, keepdims=True))
        alpha = jnp.exp(m - m_new)
        p = jnp.exp(sc - m_new)
        l_new = l * alpha + jnp.sum(p, axis=1, keepdims=True)
        vc = v_ref[0, 0, pl.ds(off, _KC), :]
        acc_new = acc * alpha + jnp.dot(p.astype(jnp.bfloat16), vc,
                                        preferred_element_type=jnp.float32)
        return m_new, l_new, acc_new

    m0 = jnp.full((_C, 1), -1e30, jnp.float32)
    l0 = jnp.zeros((_C, 1), jnp.float32)
    a0 = jnp.zeros((_C, _DH), jnp.float32)
    m, l, acc = jax.lax.fori_loop(0, qc + 1, body, (m0, l0, a0))
    seq = acc / l

    pw = pw_ref[0, 0]
    a = jnp.dot(pw, ea_ref[...], preferred_element_type=jnp.float32)
    b = jnp.dot(pw, eb_ref[...], preferred_element_type=jnp.float32)
    c = jnp.dot(pw, ec_ref[...], preferred_element_type=jnp.float32)
    d = jnp.dot(pw, ed_ref[...], preferred_element_type=jnp.float32)
    parts = a * b - c * d
    s2 = jnp.sum(parts * parts, axis=1, keepdims=True)
    nr = jnp.maximum(jnp.sqrt(s2), 1e-12)
    wl = parts / nr
    u = jnp.dot(wl, eti_ref[...], preferred_element_type=jnp.float32)
    v = jnp.dot(wl, etj_ref[...], preferred_element_type=jnp.float32)
    o = u * v

    carry_s = s_ref[...]
    dp = jnp.exp(jax.lax.broadcasted_iota(jnp.int32, (_C, _NGP), 0).astype(jnp.float32) * _LN_DECAY)
    gf = dp * carry_s + jnp.dot(l_ref[0:_C, :], o, preferred_element_type=jnp.float32)
    s_ref[...] = _DECAY_C * carry_s + jnp.dot(l_ref[_C:_C + 1, :], o,
                                              preferred_element_type=jnp.float32)

    pre = jnp.dot(gf, m1_ref[...], preferred_element_type=jnp.float32) + m1b_ref[...]
    h1 = 0.5 * pre * (1.0 + jax.lax.erf(pre * 0.7071067811865476))
    mem = jnp.dot(h1, m2_ref[...], preferred_element_type=jnp.float32) + m2b_ref[...]

    oh = (jax.lax.broadcasted_iota(jnp.int32, (_C, _H), 1) == h).astype(jnp.float32)
    gcol = jnp.sum(gl_ref[0] * oh, axis=1, keepdims=True)
    gate = jax.nn.sigmoid(gcol)

    out_ref[0, 0] = seq + gate * mem


def _out_kernel(c_ref, w_ref, b_ref, o_ref):
    o_ref[...] = jnp.dot(c_ref[...], w_ref[...], preferred_element_type=jnp.float32) + b_ref[...]


def kernel(x, qkv_w, qkv_b, w1_w, w2_w, mlp1_w, mlp1_b, mlp2_w, mlp2_b,
           gate_w, gate_b, out_w, out_b):
    bsz, t, dm = x.shape
    f32 = jnp.float32
    xf = x.reshape(bsz * t, dm).astype(jnp.bfloat16)
    rows = bsz * t
    ngrid = rows // _RC
    bf16 = jnp.bfloat16

    qkv, p1, p2, glog = pl.pallas_call(
        _proj_kernel,
        grid=(ngrid,),
        in_specs=[
            pl.BlockSpec((_RC, dm), lambda i: (i, 0)),
            pl.BlockSpec((dm, 3 * dm), lambda i: (0, 0)),
            pl.BlockSpec((1, 3 * dm), lambda i: (0, 0)),
            pl.BlockSpec((dm, _H * _P), lambda i: (0, 0)),
            pl.BlockSpec((dm, _H * _P), lambda i: (0, 0)),
            pl.BlockSpec((dm, _H), lambda i: (0, 0)),
            pl.BlockSpec((1, _H), lambda i: (0, 0)),
        ],
        out_specs=[
            pl.BlockSpec((_RC, 3 * dm), lambda i: (i, 0)),
            pl.BlockSpec((_RC, _H * _P), lambda i: (i, 0)),
            pl.BlockSpec((_RC, _H * _P), lambda i: (i, 0)),
            pl.BlockSpec((_RC, _H), lambda i: (i, 0)),
        ],
        out_shape=[
            jax.ShapeDtypeStruct((rows, 3 * dm), bf16),
            jax.ShapeDtypeStruct((rows, _H * _P), f32),
            jax.ShapeDtypeStruct((rows, _H * _P), f32),
            jax.ShapeDtypeStruct((rows, _H), f32),
        ],
        compiler_params=pltpu.CompilerParams(
            dimension_semantics=("parallel",),
        ),
    )(xf, qkv_w.astype(bf16), qkv_b.reshape(1, -1), w1_w.astype(bf16),
      w2_w.astype(bf16), gate_w.astype(bf16), gate_b.reshape(1, -1))

    qkvh = jnp.transpose(qkv.reshape(bsz, t, 3 * _H, _DH), (0, 2, 1, 3))
    glog3 = glog.reshape(bsz, t, _H)
    p1r = p1.reshape(bsz, t, _H, _P)
    p1s = jnp.concatenate([jnp.zeros((bsz, 1, _H, _P), f32), p1r[:, :-1]], axis=1)
    p2r = p2.reshape(bsz, t, _H, _P)
    pw = jnp.concatenate([p1s, p2r], axis=-1)
    pwt = jnp.transpose(pw, (0, 2, 1, 3))

    m1p = jnp.concatenate([mlp1_w, jnp.zeros((_NGP - _NG, _DH), f32)], axis=0)

    nq = t // _C
    hh = _H

    combined = pl.pallas_call(
        _attn_gram_kernel,
        grid=(bsz * _H, nq),
        in_specs=[
            pl.BlockSpec((1, 1, _C, _DH), lambda bh, qc: (bh // hh, bh % hh, qc, 0)),
            pl.BlockSpec((1, 1, t, _DH), lambda bh, qc: (bh // hh, hh + bh % hh, 0, 0)),
            pl.BlockSpec((1, 1, t, _DH), lambda bh, qc: (bh // hh, 2 * hh + bh % hh, 0, 0)),
            pl.BlockSpec((1, 1, _C, 8), lambda bh, qc: (bh // hh, bh % hh, qc, 0)),
            pl.BlockSpec((1, _C, _H), lambda bh, qc: (bh // hh, qc, 0)),
            pl.BlockSpec((_C + 8, _C), lambda bh, qc: (0, 0)),
            pl.BlockSpec((8, 8), lambda bh, qc: (0, 0)),
            pl.BlockSpec((8, 8), lambda bh, qc: (0, 0)),
            pl.BlockSpec((8, 8), lambda bh, qc: (0, 0)),
            pl.BlockSpec((8, 8), lambda bh, qc: (0, 0)),
            pl.BlockSpec((8, _NGP), lambda bh, qc: (0, 0)),
            pl.BlockSpec((8, _NGP), lambda bh, qc: (0, 0)),
            pl.BlockSpec((_NGP, _DH), lambda bh, qc: (0, 0)),
            pl.BlockSpec((1, _DH), lambda bh, qc: (0, 0)),
            pl.BlockSpec((_DH, _DH), lambda bh, qc: (0, 0)),
            pl.BlockSpec((1, _DH), lambda bh, qc: (0, 0)),
        ],
        out_specs=pl.BlockSpec((1, 1, _C, _DH), lambda bh, qc: (bh // hh, bh % hh, qc, 0)),
        out_shape=jax.ShapeDtypeStruct((bsz, _H, t, _DH), f32),
        scratch_shapes=[pltpu.VMEM((1, _NGP), f32)],
        compiler_params=pltpu.CompilerParams(
            dimension_semantics=("parallel", "arbitrary"),
        ),
    )(qkvh, qkvh, qkvh, pwt, glog3,
      jnp.asarray(_LFULL), jnp.asarray(_EA), jnp.asarray(_EB), jnp.asarray(_EC),
      jnp.asarray(_ED), jnp.asarray(_ETI), jnp.asarray(_ETJ),
      m1p, mlp1_b.reshape(1, -1), mlp2_w, mlp2_b.reshape(1, -1))

    out = pl.pallas_call(
        _out_kernel,
        grid=(ngrid,),
        in_specs=[
            pl.BlockSpec((_RC, dm), lambda i: (i, 0)),
            pl.BlockSpec((dm, dm), lambda i: (0, 0)),
            pl.BlockSpec((1, dm), lambda i: (0, 0)),
        ],
        out_specs=pl.BlockSpec((_RC, dm), lambda i: (i, 0)),
        out_shape=jax.ShapeDtypeStruct((rows, dm), f32),
        compiler_params=pltpu.CompilerParams(
            dimension_semantics=("parallel",),
        ),
    )(jnp.transpose(combined, (0, 2, 1, 3)).reshape(rows, dm).astype(bf16),
      out_w.astype(bf16), out_b.reshape(1, -1))

    return out.reshape(bsz, t, dm)


# speedup vs baseline: 5.0280x; 1.3642x over previous
from itertools import combinations

import numpy as np
import jax
import jax.numpy as jnp
from jax.experimental import pallas as pl
from jax.experimental.pallas import tpu as pltpu

_D = 1024
_H = 16
_DH = 64
_P = 4
_PD = 6
_NG = 21
_NGP = 24
_DECAY = 0.99
_C = 256
_KC = 256
_RC = 512
_SCALE = _DH ** -0.5
_LN_DECAY = float(np.log(_DECAY))
_DECAY_C = float(_DECAY ** _C)

_PAIRS = list(combinations(range(_P), 2))
_TI, _TJ = np.triu_indices(_PD)


def _np_consts():
    ea = np.zeros((8, 8), np.float32)
    eb = np.zeros((8, 8), np.float32)
    ec = np.zeros((8, 8), np.float32)
    ed = np.zeros((8, 8), np.float32)
    for kk, (i, j) in enumerate(_PAIRS):
        ea[kk, i] = 1.0
        eb[kk, 4 + j] = 1.0
        ec[kk, j] = 1.0
        ed[kk, 4 + i] = 1.0
    eti = np.zeros((_NGP, 8), np.float32)
    etj = np.zeros((_NGP, 8), np.float32)
    for kk in range(_NG):
        eti[kk, _TI[kk]] = 1.0
        etj[kk, _TJ[kk]] = 1.0
    i = np.arange(_C)[:, None]
    s = np.arange(_C)[None, :]
    lmat = np.where(s < i, _DECAY ** np.maximum(i - 1 - s, 0), 0.0).astype(np.float32)
    lt = np.ascontiguousarray(lmat.T)
    dvec = (_DECAY ** (_C - 1 - np.arange(_C))).astype(np.float32).reshape(1, _C)
    return ea, eb, ec, ed, eti, etj, lt, dvec


_EA, _EB, _EC, _ED, _ETI, _ETJ, _LT, _DVEC = _np_consts()


def _proj_kernel(x_ref, qw_ref, qb_ref, w1_ref, w2_ref, gw_ref, gb_ref,
                 qkv_ref, p1_ref, p2_ref, gl_ref):
    xb = x_ref[...]
    cdims = (((0,), (0,)), ((), ()))
    qkv = jax.lax.dot_general(qw_ref[...], xb, cdims,
                              preferred_element_type=jnp.float32) + qb_ref[...]
    qkv_ref[...] = qkv.astype(jnp.bfloat16)
    p1_ref[...] = jax.lax.dot_general(w1_ref[...], xb, cdims,
                                      preferred_element_type=jnp.float32)
    p2_ref[...] = jax.lax.dot_general(w2_ref[...], xb, cdims,
                                      preferred_element_type=jnp.float32)
    gl_ref[...] = jax.lax.dot_general(gw_ref[...], xb, cdims,
                                      preferred_element_type=jnp.float32) + gb_ref[...]


def _attn_gram_kernel(q_ref, k_ref, v_ref, pw_ref, gl_ref, lt_ref, dv_ref,
                      ea_ref, eb_ref, ec_ref, ed_ref, eti_ref, etj_ref,
                      m1_ref, m1b_ref, m2_ref, m2b_ref,
                      out_ref, s_ref):
    bh = pl.program_id(0)
    qc = pl.program_id(1)
    h = bh % _H
    t0 = qc * _C
    f32 = jnp.float32
    bf16 = jnp.bfloat16

    @pl.when(qc == 0)
    def _():
        s_ref[...] = jnp.zeros((_NGP, 1), f32)

    qt = q_ref[0]
    cdA = (((0,), (0,)), ((), ()))
    cdS = (((1,), (0,)), ((), ()))

    def body(j, carry):
        m, l, acc = carry
        off = pl.multiple_of(j * _KC, _KC)
        kc = k_ref[0, :, pl.ds(off, _KC)]
        st = jax.lax.dot_general(kc, qt, cdA, preferred_element_type=f32) * _SCALE
        m_new = jnp.maximum(m, jnp.max(st, axis=0, keepdims=True))
        alpha = jnp.exp(m - m_new)
        p = jnp.exp(st - m_new)
        l_new = l * alpha + jnp.sum(p, axis=0, keepdims=True)
        vc = v_ref[0, :, pl.ds(off, _KC)]
        acc_new = acc * alpha + jax.lax.dot_general(
            vc, p.astype(bf16), cdS, preferred_element_type=f32)
        return m_new, l_new, acc_new

    m0 = jnp.full((1, _C), -1e30, f32)
    l0 = jnp.zeros((1, _C), f32)
    a0 = jnp.zeros((_DH, _C), f32)
    m, l, acc = jax.lax.fori_loop(0, qc, body, (m0, l0, a0))

    kd = k_ref[0, :, pl.ds(t0, _KC)]
    st = jax.lax.dot_general(kd, qt, cdA, preferred_element_type=f32) * _SCALE
    ki = jax.lax.broadcasted_iota(jnp.int32, (_KC, _C), 0)
    qi = jax.lax.broadcasted_iota(jnp.int32, (_KC, _C), 1)
    st = jnp.where(ki > qi, -1e30, st)
    m_new = jnp.maximum(m, jnp.max(st, axis=0, keepdims=True))
    alpha = jnp.exp(m - m_new)
    p = jnp.exp(st - m_new)
    l = l * alpha + jnp.sum(p, axis=0, keepdims=True)
    vd = v_ref[0, :, pl.ds(t0, _KC)]
    acc = acc * alpha + jax.lax.dot_general(vd, p.astype(bf16), cdS,
                                            preferred_element_type=f32)
    seq = acc / l

    pw = pw_ref[0]
    a = jnp.dot(ea_ref[...], pw, preferred_element_type=f32)
    b = jnp.dot(eb_ref[...], pw, preferred_element_type=f32)
    c = jnp.dot(ec_ref[...], pw, preferred_element_type=f32)
    d = jnp.dot(ed_ref[...], pw, preferred_element_type=f32)
    parts = a * b - c * d
    s2 = jnp.sum(parts * parts, axis=0, keepdims=True)
    nr = jnp.maximum(jnp.sqrt(s2), 1e-12)
    wl = parts / nr
    u = jnp.dot(eti_ref[...], wl, preferred_element_type=f32)
    v = jnp.dot(etj_ref[...], wl, preferred_element_type=f32)
    o = u * v

    carry_s = s_ref[...]
    dp = jnp.exp(jax.lax.broadcasted_iota(jnp.int32, (_NGP, _C), 1).astype(f32)
                 * _LN_DECAY)
    gf = dp * carry_s + jnp.dot(o, lt_ref[...], preferred_element_type=f32)
    s_ref[...] = _DECAY_C * carry_s + jnp.sum(o * dv_ref[...], axis=1, keepdims=True)

    pre = jnp.dot(m1_ref[...], gf, preferred_element_type=f32) + m1b_ref[...]
    h1 = 0.5 * pre * (1.0 + jax.lax.erf(pre * 0.7071067811865476))
    mem = jnp.dot(m2_ref[...], h1, preferred_element_type=f32) + m2b_ref[...]

    oh = (jax.lax.broadcasted_iota(jnp.int32, (_H, 1), 0) == h).astype(f32)
    grow = jnp.sum(gl_ref[...] * oh, axis=0, keepdims=True)
    gate = jax.nn.sigmoid(grow)

    out_ref[0] = (seq + gate * mem).astype(bf16)


def _out_kernel(c_ref, w_ref, b_ref, o_ref):
    o_ref[...] = jax.lax.dot_general(
        c_ref[...], w_ref[...], (((0,), (0,)), ((), ())),
        preferred_element_type=jnp.float32) + b_ref[...]


def kernel(x, qkv_w, qkv_b, w1_w, w2_w, mlp1_w, mlp1_b, mlp2_w, mlp2_b,
           gate_w, gate_b, out_w, out_b):
    bsz, t, dm = x.shape
    f32 = jnp.float32
    bf16 = jnp.bfloat16
    rows = bsz * t
    ngrid = rows // _RC
    nq = t // _C
    hh = _H

    xt = jnp.transpose(x.reshape(rows, dm).astype(bf16))

    qkvt, p1t, p2t, glt = pl.pallas_call(
        _proj_kernel,
        grid=(ngrid,),
        in_specs=[
            pl.BlockSpec((dm, _RC), lambda i: (0, i)),
            pl.BlockSpec((dm, 3 * dm), lambda i: (0, 0)),
            pl.BlockSpec((3 * dm, 1), lambda i: (0, 0)),
            pl.BlockSpec((dm, _H * _P), lambda i: (0, 0)),
            pl.BlockSpec((dm, _H * _P), lambda i: (0, 0)),
            pl.BlockSpec((dm, _H), lambda i: (0, 0)),
            pl.BlockSpec((_H, 1), lambda i: (0, 0)),
        ],
        out_specs=[
            pl.BlockSpec((3 * dm, _RC), lambda i: (0, i)),
            pl.BlockSpec((_H * _P, _RC), lambda i: (0, i)),
            pl.BlockSpec((_H * _P, _RC), lambda i: (0, i)),
            pl.BlockSpec((_H, _RC), lambda i: (0, i)),
        ],
        out_shape=[
            jax.ShapeDtypeStruct((3 * dm, rows), bf16),
            jax.ShapeDtypeStruct((_H * _P, rows), f32),
            jax.ShapeDtypeStruct((_H * _P, rows), f32),
            jax.ShapeDtypeStruct((_H, rows), f32),
        ],
        compiler_params=pltpu.CompilerParams(
            dimension_semantics=("parallel",),
        ),
    )(xt, qkv_w.astype(bf16), qkv_b.reshape(-1, 1), w1_w.astype(bf16),
      w2_w.astype(bf16), gate_w.astype(bf16), gate_b.reshape(-1, 1))

    qkvh = qkvt.reshape(3 * _H, _DH, rows)
    p1b = p1t.reshape(_H, _P, bsz, t)
    p1s = jnp.concatenate([jnp.zeros((_H, _P, bsz, 1), f32), p1b[..., :-1]], axis=3)
    p2b = p2t.reshape(_H, _P, bsz, t)
    pwt = jnp.concatenate([p1s, p2b], axis=1).reshape(_H, 8, rows)

    m1tp = jnp.concatenate([mlp1_w.T, jnp.zeros((_DH, _NGP - _NG), f32)], axis=1)

    combined_t = pl.pallas_call(
        _attn_gram_kernel,
        grid=(bsz * _H, nq),
        in_specs=[
            pl.BlockSpec((1, _DH, _C), lambda bh, qc: (bh % hh, 0, (bh // hh) * nq + qc)),
            pl.BlockSpec((1, _DH, t), lambda bh, qc: (hh + bh % hh, 0, bh // hh)),
            pl.BlockSpec((1, _DH, t), lambda bh, qc: (2 * hh + bh % hh, 0, bh // hh)),
            pl.BlockSpec((1, 8, _C), lambda bh, qc: (bh % hh, 0, (bh // hh) * nq + qc)),
            pl.BlockSpec((_H, _C), lambda bh, qc: (0, (bh // hh) * nq + qc)),
            pl.BlockSpec((_C, _C), lambda bh, qc: (0, 0)),
            pl.BlockSpec((1, _C), lambda bh, qc: (0, 0)),
            pl.BlockSpec((8, 8), lambda bh, qc: (0, 0)),
            pl.BlockSpec((8, 8), lambda bh, qc: (0, 0)),
            pl.BlockSpec((8, 8), lambda bh, qc: (0, 0)),
            pl.BlockSpec((8, 8), lambda bh, qc: (0, 0)),
            pl.BlockSpec((_NGP, 8), lambda bh, qc: (0, 0)),
            pl.BlockSpec((_NGP, 8), lambda bh, qc: (0, 0)),
            pl.BlockSpec((_DH, _NGP), lambda bh, qc: (0, 0)),
            pl.BlockSpec((_DH, 1), lambda bh, qc: (0, 0)),
            pl.BlockSpec((_DH, _DH), lambda bh, qc: (0, 0)),
            pl.BlockSpec((_DH, 1), lambda bh, qc: (0, 0)),
        ],
        out_specs=pl.BlockSpec((1, _DH, _C), lambda bh, qc: (bh % hh, 0, (bh // hh) * nq + qc)),
        out_shape=jax.ShapeDtypeStruct((_H, _DH, rows), bf16),
        scratch_shapes=[pltpu.VMEM((_NGP, 1), f32)],
        compiler_params=pltpu.CompilerParams(
            dimension_semantics=("parallel", "arbitrary"),
        ),
    )(qkvh, qkvh, qkvh, pwt, glt,
      jnp.asarray(_LT), jnp.asarray(_DVEC), jnp.asarray(_EA), jnp.asarray(_EB),
      jnp.asarray(_EC), jnp.asarray(_ED), jnp.asarray(_ETI), jnp.asarray(_ETJ),
      m1tp.astype(f32), mlp1_b.reshape(-1, 1), mlp2_w.T, mlp2_b.reshape(-1, 1))

    out = pl.pallas_call(
        _out_kernel,
        grid=(ngrid,),
        in_specs=[
            pl.BlockSpec((dm, _RC), lambda i: (0, i)),
            pl.BlockSpec((dm, dm), lambda i: (0, 0)),
            pl.BlockSpec((1, dm), lambda i: (0, 0)),
        ],
        out_specs=pl.BlockSpec((_RC, dm), lambda i: (i, 0)),
        out_shape=jax.ShapeDtypeStruct((rows, dm), f32),
        compiler_params=pltpu.CompilerParams(
            dimension_semantics=("parallel",),
        ),
    )(combined_t.reshape(dm, rows), out_w.astype(bf16), out_b.reshape(1, -1))

    return out.reshape(bsz, t, dm)


# speedup vs baseline: 5.8595x; 1.1654x over previous
from itertools import combinations

import numpy as np
import jax
import jax.numpy as jnp
from jax.experimental import pallas as pl
from jax.experimental.pallas import tpu as pltpu

_D = 1024
_H = 16
_DH = 64
_P = 4
_PD = 6
_NG = 21
_NGP = 24
_DECAY = 0.99
_C = 256
_KC = 256
_RC = 512
_G = 2
_SCALE = _DH ** -0.5
_LN_DECAY = float(np.log(_DECAY))
_DECAY_C = float(_DECAY ** _C)

_PAIRS = list(combinations(range(_P), 2))
_TI, _TJ = np.triu_indices(_PD)


def _np_consts():
    ea = np.zeros((8, 8), np.float32)
    eb = np.zeros((8, 8), np.float32)
    ec = np.zeros((8, 8), np.float32)
    ed = np.zeros((8, 8), np.float32)
    for kk, (i, j) in enumerate(_PAIRS):
        ea[kk, i] = 1.0
        eb[kk, 4 + j] = 1.0
        ec[kk, j] = 1.0
        ed[kk, 4 + i] = 1.0
    eti = np.zeros((_NGP, 8), np.float32)
    etj = np.zeros((_NGP, 8), np.float32)
    for kk in range(_NG):
        eti[kk, _TI[kk]] = 1.0
        etj[kk, _TJ[kk]] = 1.0
    i = np.arange(_C)[:, None]
    s = np.arange(_C)[None, :]
    lmat = np.where(s < i, _DECAY ** np.maximum(i - 1 - s, 0), 0.0).astype(np.float32)
    lt = np.ascontiguousarray(lmat.T)
    dvec = (_DECAY ** (_C - 1 - np.arange(_C))).astype(np.float32).reshape(1, _C)
    return ea, eb, ec, ed, eti, etj, lt, dvec


_EA, _EB, _EC, _ED, _ETI, _ETJ, _LT, _DVEC = _np_consts()


def _proj_kernel(x_ref, qw_ref, qb_ref, w1_ref, w2_ref, gw_ref, gb_ref,
                 qkv_ref, p1_ref, p2_ref, gl_ref):
    xb = x_ref[...]
    cdims = (((0,), (0,)), ((), ()))
    qkv = jax.lax.dot_general(qw_ref[...], xb, cdims,
                              preferred_element_type=jnp.float32) + qb_ref[...]
    qkv_ref[...] = qkv.astype(jnp.bfloat16)
    p1_ref[...] = jax.lax.dot_general(w1_ref[...], xb, cdims,
                                      preferred_element_type=jnp.float32)
    p2_ref[...] = jax.lax.dot_general(w2_ref[...], xb, cdims,
                                      preferred_element_type=jnp.float32)
    gl_ref[...] = jax.lax.dot_general(gw_ref[...], xb, cdims,
                                      preferred_element_type=jnp.float32) + gb_ref[...]


def _attn_gram_kernel(q_ref, k_ref, v_ref, pw_ref, gl_ref, lt_ref, dv_ref,
                      ea_ref, eb_ref, ec_ref, ed_ref, eti_ref, etj_ref,
                      m1_ref, m1b_ref, m2_ref, m2b_ref,
                      out_ref, s_ref):
    qc = pl.program_id(1)
    t0 = qc * _C
    f32 = jnp.float32
    bf16 = jnp.bfloat16

    @pl.when(qc == 0)
    def _():
        s_ref[...] = jnp.zeros((_G * _NGP, 1), f32)

    cdA = (((0,), (0,)), ((), ()))
    cdS = (((1,), (0,)), ((), ()))

    def one_head_chunk(g, off, m, l, acc, masked):
        kc = k_ref[g, :, pl.ds(off, _KC)]
        st = jax.lax.dot_general(kc, q_ref[g], cdA,
                                 preferred_element_type=f32) * _SCALE
        if masked:
            ki = jax.lax.broadcasted_iota(jnp.int32, (_KC, _C), 0)
            qi = jax.lax.broadcasted_iota(jnp.int32, (_KC, _C), 1)
            st = jnp.where(ki > qi, -1e30, st)
        m_new = jnp.maximum(m, jnp.max(st, axis=0, keepdims=True))
        alpha = jnp.exp(m - m_new)
        p = jnp.exp(st - m_new)
        l_new = l * alpha + jnp.sum(p, axis=0, keepdims=True)
        vc = v_ref[g, :, pl.ds(off, _KC)]
        acc_new = acc * alpha + jax.lax.dot_general(
            vc, p.astype(bf16), cdS, preferred_element_type=f32)
        return m_new, l_new, acc_new

    def body(j, carry):
        off = pl.multiple_of(j * _KC, _KC)
        return tuple(one_head_chunk(g, off, *carry[g], masked=False)
                     for g in range(_G))

    init = tuple((jnp.full((1, _C), -1e30, f32), jnp.zeros((1, _C), f32),
                  jnp.zeros((_DH, _C), f32)) for _ in range(_G))
    carry = jax.lax.fori_loop(0, qc, body, init)
    seqs = []
    for g in range(_G):
        m, l, acc = one_head_chunk(g, t0, *carry[g], masked=True)
        seqs.append(acc / l)

    dp = jnp.exp(jax.lax.broadcasted_iota(jnp.int32, (_NGP, _C), 1).astype(f32)
                 * _LN_DECAY)
    for g in range(_G):
        pw = pw_ref[g]
        a = jnp.dot(ea_ref[...], pw, preferred_element_type=f32)
        b = jnp.dot(eb_ref[...], pw, preferred_element_type=f32)
        c = jnp.dot(ec_ref[...], pw, preferred_element_type=f32)
        d = jnp.dot(ed_ref[...], pw, preferred_element_type=f32)
        parts = a * b - c * d
        s2 = jnp.sum(parts * parts, axis=0, keepdims=True)
        nr = jnp.maximum(jnp.sqrt(s2), 1e-12)
        wl = parts / nr
        u = jnp.dot(eti_ref[...], wl, preferred_element_type=f32)
        v = jnp.dot(etj_ref[...], wl, preferred_element_type=f32)
        o = u * v

        carry_s = s_ref[g * _NGP:(g + 1) * _NGP, :]
        gf = dp * carry_s + jnp.dot(o, lt_ref[...], preferred_element_type=f32)
        s_ref[g * _NGP:(g + 1) * _NGP, :] = (
            _DECAY_C * carry_s + jnp.sum(o * dv_ref[...], axis=1, keepdims=True))

        pre = jnp.dot(m1_ref[...], gf, preferred_element_type=f32) + m1b_ref[...]
        h1 = 0.5 * pre * (1.0 + jax.lax.erf(pre * 0.7071067811865476))
        mem = jnp.dot(m2_ref[...], h1, preferred_element_type=f32) + m2b_ref[...]

        gate = jax.nn.sigmoid(gl_ref[0, g:g + 1, :])
        out_ref[g] = (seqs[g] + gate * mem).astype(bf16)


def _out_kernel(c_ref, w_ref, b_ref, o_ref):
    o_ref[...] = jax.lax.dot_general(
        c_ref[...], w_ref[...], (((0,), (0,)), ((), ())),
        preferred_element_type=jnp.float32) + b_ref[...]


def kernel(x, qkv_w, qkv_b, w1_w, w2_w, mlp1_w, mlp1_b, mlp2_w, mlp2_b,
           gate_w, gate_b, out_w, out_b):
    bsz, t, dm = x.shape
    f32 = jnp.float32
    bf16 = jnp.bfloat16
    rows = bsz * t
    ngrid = rows // _RC
    nq = t // _C
    hh = _H

    xt = jnp.transpose(x.reshape(rows, dm).astype(bf16))

    qkvt, p1t, p2t, glt = pl.pallas_call(
        _proj_kernel,
        grid=(ngrid,),
        in_specs=[
            pl.BlockSpec((dm, _RC), lambda i: (0, i)),
            pl.BlockSpec((dm, 3 * dm), lambda i: (0, 0)),
            pl.BlockSpec((3 * dm, 1), lambda i: (0, 0)),
            pl.BlockSpec((dm, _H * _P), lambda i: (0, 0)),
            pl.BlockSpec((dm, _H * _P), lambda i: (0, 0)),
            pl.BlockSpec((dm, _H), lambda i: (0, 0)),
            pl.BlockSpec((_H, 1), lambda i: (0, 0)),
        ],
        out_specs=[
            pl.BlockSpec((3 * dm, _RC), lambda i: (0, i)),
            pl.BlockSpec((_H * _P, _RC), lambda i: (0, i)),
            pl.BlockSpec((_H * _P, _RC), lambda i: (0, i)),
            pl.BlockSpec((_H, _RC), lambda i: (0, i)),
        ],
        out_shape=[
            jax.ShapeDtypeStruct((3 * dm, rows), bf16),
            jax.ShapeDtypeStruct((_H * _P, rows), f32),
            jax.ShapeDtypeStruct((_H * _P, rows), f32),
            jax.ShapeDtypeStruct((_H, rows), f32),
        ],
        compiler_params=pltpu.CompilerParams(
            dimension_semantics=("parallel",),
        ),
    )(xt, qkv_w.astype(bf16), qkv_b.reshape(-1, 1), w1_w.astype(bf16),
      w2_w.astype(bf16), gate_w.astype(bf16), gate_b.reshape(-1, 1))

    qkvh = qkvt.reshape(3 * _H, _DH, rows)
    p1b = p1t.reshape(_H, _P, bsz, t)
    p1s = jnp.concatenate([jnp.zeros((_H, _P, bsz, 1), f32), p1b[..., :-1]], axis=3)
    p2b = p2t.reshape(_H, _P, bsz, t)
    pwt = jnp.concatenate([p1s, p2b], axis=1).reshape(_H, 8, rows)

    m1tp = jnp.concatenate([mlp1_w.T, jnp.zeros((_DH, _NGP - _NG), f32)], axis=1)

    hgn = _H // _G
    combined_t = pl.pallas_call(
        _attn_gram_kernel,
        grid=(bsz * hgn, nq),
        in_specs=[
            pl.BlockSpec((_G, _DH, _C), lambda bh, qc: (bh % hgn, 0, (bh // hgn) * nq + qc)),
            pl.BlockSpec((_G, _DH, t), lambda bh, qc: (hgn + bh % hgn, 0, bh // hgn)),
            pl.BlockSpec((_G, _DH, t), lambda bh, qc: (2 * hgn + bh % hgn, 0, bh // hgn)),
            pl.BlockSpec((_G, 8, _C), lambda bh, qc: (bh % hgn, 0, (bh // hgn) * nq + qc)),
            pl.BlockSpec((1, _G, _C), lambda bh, qc: (bh % hgn, 0, (bh // hgn) * nq + qc)),
            pl.BlockSpec((_C, _C), lambda bh, qc: (0, 0)),
            pl.BlockSpec((1, _C), lambda bh, qc: (0, 0)),
            pl.BlockSpec((8, 8), lambda bh, qc: (0, 0)),
            pl.BlockSpec((8, 8), lambda bh, qc: (0, 0)),
            pl.BlockSpec((8, 8), lambda bh, qc: (0, 0)),
            pl.BlockSpec((8, 8), lambda bh, qc: (0, 0)),
            pl.BlockSpec((_NGP, 8), lambda bh, qc: (0, 0)),
            pl.BlockSpec((_NGP, 8), lambda bh, qc: (0, 0)),
            pl.BlockSpec((_DH, _NGP), lambda bh, qc: (0, 0)),
            pl.BlockSpec((_DH, 1), lambda bh, qc: (0, 0)),
            pl.BlockSpec((_DH, _DH), lambda bh, qc: (0, 0)),
            pl.BlockSpec((_DH, 1), lambda bh, qc: (0, 0)),
        ],
        out_specs=pl.BlockSpec((_G, _DH, _C), lambda bh, qc: (bh % hgn, 0, (bh // hgn) * nq + qc)),
        out_shape=jax.ShapeDtypeStruct((_H, _DH, rows), bf16),
        scratch_shapes=[pltpu.VMEM((_G * _NGP, 1), f32)],
        compiler_params=pltpu.CompilerParams(
            dimension_semantics=("parallel", "arbitrary"),
        ),
    )(qkvh, qkvh, qkvh, pwt, glt.reshape(hgn, _G, rows),
      jnp.asarray(_LT), jnp.asarray(_DVEC), jnp.asarray(_EA), jnp.asarray(_EB),
      jnp.asarray(_EC), jnp.asarray(_ED), jnp.asarray(_ETI), jnp.asarray(_ETJ),
      m1tp.astype(f32), mlp1_b.reshape(-1, 1), mlp2_w.T, mlp2_b.reshape(-1, 1))

    out = pl.pallas_call(
        _out_kernel,
        grid=(ngrid,),
        in_specs=[
            pl.BlockSpec((dm, _RC), lambda i: (0, i)),
            pl.BlockSpec((dm, dm), lambda i: (0, 0)),
            pl.BlockSpec((1, dm), lambda i: (0, 0)),
        ],
        out_specs=pl.BlockSpec((_RC, dm), lambda i: (i, 0)),
        out_shape=jax.ShapeDtypeStruct((rows, dm), f32),
        compiler_params=pltpu.CompilerParams(
            dimension_semantics=("parallel",),
        ),
    )(combined_t.reshape(dm, rows), out_w.astype(bf16), out_b.reshape(1, -1))

    return out.reshape(bsz, t, dm)


# speedup vs baseline: 6.4642x; 1.1032x over previous
from itertools import combinations

import numpy as np
import jax
import jax.numpy as jnp
from jax.experimental import pallas as pl
from jax.experimental.pallas import tpu as pltpu

_D = 1024
_H = 16
_DH = 64
_P = 4
_PD = 6
_NG = 21
_NGP = 24
_DECAY = 0.99
_C = 256
_KC = 256
_RC = 512
_G = 4
_SCALE = _DH ** -0.5
_LN_DECAY = float(np.log(_DECAY))
_DECAY_C = float(_DECAY ** _C)

_PAIRS = list(combinations(range(_P), 2))
_TI, _TJ = np.triu_indices(_PD)


def _np_consts():
    ea = np.zeros((8, 8), np.float32)
    eb = np.zeros((8, 8), np.float32)
    ec = np.zeros((8, 8), np.float32)
    ed = np.zeros((8, 8), np.float32)
    for kk, (i, j) in enumerate(_PAIRS):
        ea[kk, i] = 1.0
        eb[kk, 4 + j] = 1.0
        ec[kk, j] = 1.0
        ed[kk, 4 + i] = 1.0
    eti = np.zeros((_NGP, 8), np.float32)
    etj = np.zeros((_NGP, 8), np.float32)
    for kk in range(_NG):
        eti[kk, _TI[kk]] = 1.0
        etj[kk, _TJ[kk]] = 1.0
    i = np.arange(_C)[:, None]
    s = np.arange(_C)[None, :]
    lmat = np.where(s < i, _DECAY ** np.maximum(i - 1 - s, 0), 0.0).astype(np.float32)
    lt = np.ascontiguousarray(lmat.T)
    dvec = (_DECAY ** (_C - 1 - np.arange(_C))).astype(np.float32).reshape(1, _C)
    return ea, eb, ec, ed, eti, etj, lt, dvec


_EA, _EB, _EC, _ED, _ETI, _ETJ, _LT, _DVEC = _np_consts()


def _proj_kernel(x_ref, qw_ref, qb_ref, w1_ref, w2_ref, gw_ref, gb_ref,
                 qkv_ref, p1_ref, p2_ref, gl_ref):
    xb = x_ref[...]
    cdims = (((0,), (0,)), ((), ()))
    qkv = jax.lax.dot_general(qw_ref[...], xb, cdims,
                              preferred_element_type=jnp.float32) + qb_ref[...]
    qkv_ref[...] = qkv.astype(jnp.bfloat16)
    p1_ref[...] = jax.lax.dot_general(w1_ref[...], xb, cdims,
                                      preferred_element_type=jnp.float32)
    p2_ref[...] = jax.lax.dot_general(w2_ref[...], xb, cdims,
                                      preferred_element_type=jnp.float32)
    gl_ref[...] = jax.lax.dot_general(gw_ref[...], xb, cdims,
                                      preferred_element_type=jnp.float32) + gb_ref[...]


def _attn_gram_kernel(q_ref, k_ref, v_ref, pw_ref, gl_ref, lt_ref, dv_ref,
                      ea_ref, eb_ref, ec_ref, ed_ref, eti_ref, etj_ref,
                      m1_ref, m1b_ref, m2_ref, m2b_ref,
                      out_ref, s_ref):
    qc = pl.program_id(1)
    t0 = qc * _C
    f32 = jnp.float32
    bf16 = jnp.bfloat16

    @pl.when(qc == 0)
    def _():
        s_ref[...] = jnp.zeros((_G * _NGP, 1), f32)

    cdA = (((0,), (0,)), ((), ()))
    cdS = (((1,), (0,)), ((), ()))

    def one_head_chunk(g, off, m, l, acc, masked):
        kc = k_ref[g, :, pl.ds(off, _KC)]
        st = jax.lax.dot_general(kc, q_ref[g], cdA,
                                 preferred_element_type=f32) * _SCALE
        if masked:
            ki = jax.lax.broadcasted_iota(jnp.int32, (_KC, _C), 0)
            qi = jax.lax.broadcasted_iota(jnp.int32, (_KC, _C), 1)
            st = jnp.where(ki > qi, -1e30, st)
        m_new = jnp.maximum(m, jnp.max(st, axis=0, keepdims=True))
        alpha = jnp.exp(m - m_new)
        p = jnp.exp(st - m_new)
        l_new = l * alpha + jnp.sum(p, axis=0, keepdims=True)
        vc = v_ref[g, :, pl.ds(off, _KC)]
        acc_new = acc * alpha + jax.lax.dot_general(
            vc, p.astype(bf16), cdS, preferred_element_type=f32)
        return m_new, l_new, acc_new

    def body(j, carry):
        off = pl.multiple_of(j * _KC, _KC)
        return tuple(one_head_chunk(g, off, *carry[g], masked=False)
                     for g in range(_G))

    init = tuple((jnp.full((1, _C), -1e30, f32), jnp.zeros((1, _C), f32),
                  jnp.zeros((_DH, _C), f32)) for _ in range(_G))
    carry = jax.lax.fori_loop(0, qc, body, init)
    seqs = []
    for g in range(_G):
        m, l, acc = one_head_chunk(g, t0, *carry[g], masked=True)
        seqs.append(acc / l)

    dp = jnp.exp(jax.lax.broadcasted_iota(jnp.int32, (_NGP, _C), 1).astype(f32)
                 * _LN_DECAY)
    for g in range(_G):
        pw = pw_ref[g]
        a = jnp.dot(ea_ref[...], pw, preferred_element_type=f32)
        b = jnp.dot(eb_ref[...], pw, preferred_element_type=f32)
        c = jnp.dot(ec_ref[...], pw, preferred_element_type=f32)
        d = jnp.dot(ed_ref[...], pw, preferred_element_type=f32)
        parts = a * b - c * d
        s2 = jnp.sum(parts * parts, axis=0, keepdims=True)
        nr = jnp.maximum(jnp.sqrt(s2), 1e-12)
        wl = parts / nr
        u = jnp.dot(eti_ref[...], wl, preferred_element_type=f32)
        v = jnp.dot(etj_ref[...], wl, preferred_element_type=f32)
        o = u * v

        carry_s = s_ref[g * _NGP:(g + 1) * _NGP, :]
        gf = dp * carry_s + jnp.dot(o, lt_ref[...], preferred_element_type=f32)
        s_ref[g * _NGP:(g + 1) * _NGP, :] = (
            _DECAY_C * carry_s + jnp.sum(o * dv_ref[...], axis=1, keepdims=True))

        pre = jnp.dot(m1_ref[...], gf, preferred_element_type=f32) + m1b_ref[...]
        h1 = 0.5 * pre * (1.0 + jax.lax.erf(pre * 0.7071067811865476))
        mem = jnp.dot(m2_ref[...], h1, preferred_element_type=f32) + m2b_ref[...]

        gate = jax.nn.sigmoid(gl_ref[0, g:g + 1, :])
        out_ref[g] = (seqs[g] + gate * mem).astype(bf16)


def _out_kernel(c_ref, w_ref, b_ref, o_ref):
    o_ref[...] = jax.lax.dot_general(
        c_ref[...], w_ref[...], (((0,), (0,)), ((), ())),
        preferred_element_type=jnp.float32) + b_ref[...]


def kernel(x, qkv_w, qkv_b, w1_w, w2_w, mlp1_w, mlp1_b, mlp2_w, mlp2_b,
           gate_w, gate_b, out_w, out_b):
    bsz, t, dm = x.shape
    f32 = jnp.float32
    bf16 = jnp.bfloat16
    rows = bsz * t
    ngrid = rows // _RC
    nq = t // _C
    hh = _H

    xt = jnp.transpose(x.reshape(rows, dm).astype(bf16))

    qkvt, p1t, p2t, glt = pl.pallas_call(
        _proj_kernel,
        grid=(ngrid,),
        in_specs=[
            pl.BlockSpec((dm, _RC), lambda i: (0, i)),
            pl.BlockSpec((dm, 3 * dm), lambda i: (0, 0)),
            pl.BlockSpec((3 * dm, 1), lambda i: (0, 0)),
            pl.BlockSpec((dm, _H * _P), lambda i: (0, 0)),
            pl.BlockSpec((dm, _H * _P), lambda i: (0, 0)),
            pl.BlockSpec((dm, _H), lambda i: (0, 0)),
            pl.BlockSpec((_H, 1), lambda i: (0, 0)),
        ],
        out_specs=[
            pl.BlockSpec((3 * dm, _RC), lambda i: (0, i)),
            pl.BlockSpec((_H * _P, _RC), lambda i: (0, i)),
            pl.BlockSpec((_H * _P, _RC), lambda i: (0, i)),
            pl.BlockSpec((_H, _RC), lambda i: (0, i)),
        ],
        out_shape=[
            jax.ShapeDtypeStruct((3 * dm, rows), bf16),
            jax.ShapeDtypeStruct((_H * _P, rows), f32),
            jax.ShapeDtypeStruct((_H * _P, rows), f32),
            jax.ShapeDtypeStruct((_H, rows), f32),
        ],
        compiler_params=pltpu.CompilerParams(
            dimension_semantics=("parallel",),
        ),
    )(xt, qkv_w.astype(bf16), qkv_b.reshape(-1, 1), w1_w.astype(bf16),
      w2_w.astype(bf16), gate_w.astype(bf16), gate_b.reshape(-1, 1))

    qkvh = qkvt.reshape(3 * _H, _DH, rows)
    p1b = p1t.reshape(_H, _P, bsz, t)
    p1s = jnp.concatenate([jnp.zeros((_H, _P, bsz, 1), f32), p1b[..., :-1]], axis=3)
    p2b = p2t.reshape(_H, _P, bsz, t)
    pwt = jnp.concatenate([p1s, p2b], axis=1).reshape(_H, 8, rows)

    m1tp = jnp.concatenate([mlp1_w.T, jnp.zeros((_DH, _NGP - _NG), f32)], axis=1)

    hgn = _H // _G
    combined_t = pl.pallas_call(
        _attn_gram_kernel,
        grid=(bsz * hgn, nq),
        in_specs=[
            pl.BlockSpec((_G, _DH, _C), lambda bh, qc: (bh % hgn, 0, (bh // hgn) * nq + qc)),
            pl.BlockSpec((_G, _DH, t), lambda bh, qc: (hgn + bh % hgn, 0, bh // hgn)),
            pl.BlockSpec((_G, _DH, t), lambda bh, qc: (2 * hgn + bh % hgn, 0, bh // hgn)),
            pl.BlockSpec((_G, 8, _C), lambda bh, qc: (bh % hgn, 0, (bh // hgn) * nq + qc)),
            pl.BlockSpec((1, _G, _C), lambda bh, qc: (bh % hgn, 0, (bh // hgn) * nq + qc)),
            pl.BlockSpec((_C, _C), lambda bh, qc: (0, 0)),
            pl.BlockSpec((1, _C), lambda bh, qc: (0, 0)),
            pl.BlockSpec((8, 8), lambda bh, qc: (0, 0)),
            pl.BlockSpec((8, 8), lambda bh, qc: (0, 0)),
            pl.BlockSpec((8, 8), lambda bh, qc: (0, 0)),
            pl.BlockSpec((8, 8), lambda bh, qc: (0, 0)),
            pl.BlockSpec((_NGP, 8), lambda bh, qc: (0, 0)),
            pl.BlockSpec((_NGP, 8), lambda bh, qc: (0, 0)),
            pl.BlockSpec((_DH, _NGP), lambda bh, qc: (0, 0)),
            pl.BlockSpec((_DH, 1), lambda bh, qc: (0, 0)),
            pl.BlockSpec((_DH, _DH), lambda bh, qc: (0, 0)),
            pl.BlockSpec((_DH, 1), lambda bh, qc: (0, 0)),
        ],
        out_specs=pl.BlockSpec((_G, _DH, _C), lambda bh, qc: (bh % hgn, 0, (bh // hgn) * nq + qc)),
        out_shape=jax.ShapeDtypeStruct((_H, _DH, rows), bf16),
        scratch_shapes=[pltpu.VMEM((_G * _NGP, 1), f32)],
        compiler_params=pltpu.CompilerParams(
            dimension_semantics=("parallel", "arbitrary"),
        ),
    )(qkvh, qkvh, qkvh, pwt, glt.reshape(hgn, _G, rows),
      jnp.asarray(_LT), jnp.asarray(_DVEC), jnp.asarray(_EA), jnp.asarray(_EB),
      jnp.asarray(_EC), jnp.asarray(_ED), jnp.asarray(_ETI), jnp.asarray(_ETJ),
      m1tp.astype(f32), mlp1_b.reshape(-1, 1), mlp2_w.T, mlp2_b.reshape(-1, 1))

    out = pl.pallas_call(
        _out_kernel,
        grid=(ngrid,),
        in_specs=[
            pl.BlockSpec((dm, _RC), lambda i: (0, i)),
            pl.BlockSpec((dm, dm), lambda i: (0, 0)),
            pl.BlockSpec((1, dm), lambda i: (0, 0)),
        ],
        out_specs=pl.BlockSpec((_RC, dm), lambda i: (i, 0)),
        out_shape=jax.ShapeDtypeStruct((rows, dm), f32),
        compiler_params=pltpu.CompilerParams(
            dimension_semantics=("parallel",),
        ),
    )(combined_t.reshape(dm, rows), out_w.astype(bf16), out_b.reshape(1, -1))

    return out.reshape(bsz, t, dm)


# speedup vs baseline: 6.6886x; 1.0347x over previous
from itertools import combinations

import numpy as np
import jax
import jax.numpy as jnp
from jax.experimental import pallas as pl
from jax.experimental.pallas import tpu as pltpu

_D = 1024
_H = 16
_DH = 64
_P = 4
_PD = 6
_NG = 21
_NGP = 24
_DECAY = 0.99
_C = 256
_KC = 256
_RC = 512
_G = 8
_SCALE = _DH ** -0.5
_LN_DECAY = float(np.log(_DECAY))
_DECAY_C = float(_DECAY ** _C)

_PAIRS = list(combinations(range(_P), 2))
_TI, _TJ = np.triu_indices(_PD)


def _np_consts():
    ea = np.zeros((8, 8), np.float32)
    eb = np.zeros((8, 8), np.float32)
    ec = np.zeros((8, 8), np.float32)
    ed = np.zeros((8, 8), np.float32)
    for kk, (i, j) in enumerate(_PAIRS):
        ea[kk, i] = 1.0
        eb[kk, 4 + j] = 1.0
        ec[kk, j] = 1.0
        ed[kk, 4 + i] = 1.0
    eti = np.zeros((_NGP, 8), np.float32)
    etj = np.zeros((_NGP, 8), np.float32)
    for kk in range(_NG):
        eti[kk, _TI[kk]] = 1.0
        etj[kk, _TJ[kk]] = 1.0
    i = np.arange(_C)[:, None]
    s = np.arange(_C)[None, :]
    lmat = np.where(s < i, _DECAY ** np.maximum(i - 1 - s, 0), 0.0).astype(np.float32)
    lt = np.ascontiguousarray(lmat.T)
    dvec = (_DECAY ** (_C - 1 - np.arange(_C))).astype(np.float32).reshape(1, _C)
    return ea, eb, ec, ed, eti, etj, lt, dvec


_EA, _EB, _EC, _ED, _ETI, _ETJ, _LT, _DVEC = _np_consts()


def _proj_kernel(x_ref, qw_ref, qb_ref, w1_ref, w2_ref, gw_ref, gb_ref,
                 qkv_ref, p1_ref, p2_ref, gl_ref):
    xb = x_ref[...]
    cdims = (((0,), (0,)), ((), ()))
    qkv = jax.lax.dot_general(qw_ref[...], xb, cdims,
                              preferred_element_type=jnp.float32) + qb_ref[...]
    qkv_ref[...] = qkv.astype(jnp.bfloat16)
    p1_ref[...] = jax.lax.dot_general(w1_ref[...], xb, cdims,
                                      preferred_element_type=jnp.float32)
    p2_ref[...] = jax.lax.dot_general(w2_ref[...], xb, cdims,
                                      preferred_element_type=jnp.float32)
    gl_ref[...] = jax.lax.dot_general(gw_ref[...], xb, cdims,
                                      preferred_element_type=jnp.float32) + gb_ref[...]


def _attn_gram_kernel(q_ref, k_ref, v_ref, pw_ref, gl_ref, lt_ref, dv_ref,
                      ea_ref, eb_ref, ec_ref, ed_ref, eti_ref, etj_ref,
                      m1_ref, m1b_ref, m2_ref, m2b_ref,
                      out_ref, s_ref):
    qc = pl.program_id(1)
    t0 = qc * _C
    f32 = jnp.float32
    bf16 = jnp.bfloat16

    @pl.when(qc == 0)
    def _():
        s_ref[...] = jnp.zeros((_G * _NGP, 1), f32)

    cdA = (((0,), (0,)), ((), ()))
    cdS = (((1,), (0,)), ((), ()))

    def one_head_chunk(g, off, m, l, acc, masked):
        kc = k_ref[g, :, pl.ds(off, _KC)]
        st = jax.lax.dot_general(kc, q_ref[g], cdA,
                                 preferred_element_type=f32) * _SCALE
        if masked:
            ki = jax.lax.broadcasted_iota(jnp.int32, (_KC, _C), 0)
            qi = jax.lax.broadcasted_iota(jnp.int32, (_KC, _C), 1)
            st = jnp.where(ki > qi, -1e30, st)
        m_new = jnp.maximum(m, jnp.max(st, axis=0, keepdims=True))
        alpha = jnp.exp(m - m_new)
        p = jnp.exp(st - m_new)
        l_new = l * alpha + jnp.sum(p, axis=0, keepdims=True)
        vc = v_ref[g, :, pl.ds(off, _KC)]
        acc_new = acc * alpha + jax.lax.dot_general(
            vc, p.astype(bf16), cdS, preferred_element_type=f32)
        return m_new, l_new, acc_new

    def body(j, carry):
        off = pl.multiple_of(j * _KC, _KC)
        return tuple(one_head_chunk(g, off, *carry[g], masked=False)
                     for g in range(_G))

    init = tuple((jnp.full((1, _C), -1e30, f32), jnp.zeros((1, _C), f32),
                  jnp.zeros((_DH, _C), f32)) for _ in range(_G))
    carry = jax.lax.fori_loop(0, qc, body, init)
    seqs = []
    for g in range(_G):
        m, l, acc = one_head_chunk(g, t0, *carry[g], masked=True)
        seqs.append(acc / l)

    dp = jnp.exp(jax.lax.broadcasted_iota(jnp.int32, (_NGP, _C), 1).astype(f32)
                 * _LN_DECAY)
    for g in range(_G):
        pw = pw_ref[g]
        a = jnp.dot(ea_ref[...], pw, preferred_element_type=f32)
        b = jnp.dot(eb_ref[...], pw, preferred_element_type=f32)
        c = jnp.dot(ec_ref[...], pw, preferred_element_type=f32)
        d = jnp.dot(ed_ref[...], pw, preferred_element_type=f32)
        parts = a * b - c * d
        s2 = jnp.sum(parts * parts, axis=0, keepdims=True)
        nr = jnp.maximum(jnp.sqrt(s2), 1e-12)
        wl = parts / nr
        u = jnp.dot(eti_ref[...], wl, preferred_element_type=f32)
        v = jnp.dot(etj_ref[...], wl, preferred_element_type=f32)
        o = u * v

        carry_s = s_ref[g * _NGP:(g + 1) * _NGP, :]
        gf = dp * carry_s + jnp.dot(o, lt_ref[...], preferred_element_type=f32)
        s_ref[g * _NGP:(g + 1) * _NGP, :] = (
            _DECAY_C * carry_s + jnp.sum(o * dv_ref[...], axis=1, keepdims=True))

        pre = jnp.dot(m1_ref[...], gf, preferred_element_type=f32) + m1b_ref[...]
        h1 = 0.5 * pre * (1.0 + jax.lax.erf(pre * 0.7071067811865476))
        mem = jnp.dot(m2_ref[...], h1, preferred_element_type=f32) + m2b_ref[...]

        gate = jax.nn.sigmoid(gl_ref[0, g:g + 1, :])
        out_ref[g] = (seqs[g] + gate * mem).astype(bf16)


def _out_kernel(c_ref, w_ref, b_ref, o_ref):
    o_ref[...] = jax.lax.dot_general(
        c_ref[...], w_ref[...], (((0,), (0,)), ((), ())),
        preferred_element_type=jnp.float32) + b_ref[...]


def kernel(x, qkv_w, qkv_b, w1_w, w2_w, mlp1_w, mlp1_b, mlp2_w, mlp2_b,
           gate_w, gate_b, out_w, out_b):
    bsz, t, dm = x.shape
    f32 = jnp.float32
    bf16 = jnp.bfloat16
    rows = bsz * t
    ngrid = rows // _RC
    nq = t // _C
    hh = _H

    xt = jnp.transpose(x.reshape(rows, dm).astype(bf16))

    qkvt, p1t, p2t, glt = pl.pallas_call(
        _proj_kernel,
        grid=(ngrid,),
        in_specs=[
            pl.BlockSpec((dm, _RC), lambda i: (0, i)),
            pl.BlockSpec((dm, 3 * dm), lambda i: (0, 0)),
            pl.BlockSpec((3 * dm, 1), lambda i: (0, 0)),
            pl.BlockSpec((dm, _H * _P), lambda i: (0, 0)),
            pl.BlockSpec((dm, _H * _P), lambda i: (0, 0)),
            pl.BlockSpec((dm, _H), lambda i: (0, 0)),
            pl.BlockSpec((_H, 1), lambda i: (0, 0)),
        ],
        out_specs=[
            pl.BlockSpec((3 * dm, _RC), lambda i: (0, i)),
            pl.BlockSpec((_H * _P, _RC), lambda i: (0, i)),
            pl.BlockSpec((_H * _P, _RC), lambda i: (0, i)),
            pl.BlockSpec((_H, _RC), lambda i: (0, i)),
        ],
        out_shape=[
            jax.ShapeDtypeStruct((3 * dm, rows), bf16),
            jax.ShapeDtypeStruct((_H * _P, rows), f32),
            jax.ShapeDtypeStruct((_H * _P, rows), f32),
            jax.ShapeDtypeStruct((_H, rows), f32),
        ],
        compiler_params=pltpu.CompilerParams(
            dimension_semantics=("parallel",),
        ),
    )(xt, qkv_w.astype(bf16), qkv_b.reshape(-1, 1), w1_w.astype(bf16),
      w2_w.astype(bf16), gate_w.astype(bf16), gate_b.reshape(-1, 1))

    qkvh = qkvt.reshape(3 * _H, _DH, rows)
    p1b = p1t.reshape(_H, _P, bsz, t)
    p1s = jnp.concatenate([jnp.zeros((_H, _P, bsz, 1), f32), p1b[..., :-1]], axis=3)
    p2b = p2t.reshape(_H, _P, bsz, t)
    pwt = jnp.concatenate([p1s, p2b], axis=1).reshape(_H, 8, rows)

    m1tp = jnp.concatenate([mlp1_w.T, jnp.zeros((_DH, _NGP - _NG), f32)], axis=1)

    hgn = _H // _G
    combined_t = pl.pallas_call(
        _attn_gram_kernel,
        grid=(bsz * hgn, nq),
        in_specs=[
            pl.BlockSpec((_G, _DH, _C), lambda bh, qc: (bh % hgn, 0, (bh // hgn) * nq + qc)),
            pl.BlockSpec((_G, _DH, t), lambda bh, qc: (hgn + bh % hgn, 0, bh // hgn)),
            pl.BlockSpec((_G, _DH, t), lambda bh, qc: (2 * hgn + bh % hgn, 0, bh // hgn)),
            pl.BlockSpec((_G, 8, _C), lambda bh, qc: (bh % hgn, 0, (bh // hgn) * nq + qc)),
            pl.BlockSpec((1, _G, _C), lambda bh, qc: (bh % hgn, 0, (bh // hgn) * nq + qc)),
            pl.BlockSpec((_C, _C), lambda bh, qc: (0, 0)),
            pl.BlockSpec((1, _C), lambda bh, qc: (0, 0)),
            pl.BlockSpec((8, 8), lambda bh, qc: (0, 0)),
            pl.BlockSpec((8, 8), lambda bh, qc: (0, 0)),
            pl.BlockSpec((8, 8), lambda bh, qc: (0, 0)),
            pl.BlockSpec((8, 8), lambda bh, qc: (0, 0)),
            pl.BlockSpec((_NGP, 8), lambda bh, qc: (0, 0)),
            pl.BlockSpec((_NGP, 8), lambda bh, qc: (0, 0)),
            pl.BlockSpec((_DH, _NGP), lambda bh, qc: (0, 0)),
            pl.BlockSpec((_DH, 1), lambda bh, qc: (0, 0)),
            pl.BlockSpec((_DH, _DH), lambda bh, qc: (0, 0)),
            pl.BlockSpec((_DH, 1), lambda bh, qc: (0, 0)),
        ],
        out_specs=pl.BlockSpec((_G, _DH, _C), lambda bh, qc: (bh % hgn, 0, (bh // hgn) * nq + qc)),
        out_shape=jax.ShapeDtypeStruct((_H, _DH, rows), bf16),
        scratch_shapes=[pltpu.VMEM((_G * _NGP, 1), f32)],
        compiler_params=pltpu.CompilerParams(
            dimension_semantics=("parallel", "arbitrary"),
        ),
    )(qkvh, qkvh, qkvh, pwt, glt.reshape(hgn, _G, rows),
      jnp.asarray(_LT), jnp.asarray(_DVEC), jnp.asarray(_EA), jnp.asarray(_EB),
      jnp.asarray(_EC), jnp.asarray(_ED), jnp.asarray(_ETI), jnp.asarray(_ETJ),
      m1tp.astype(f32), mlp1_b.reshape(-1, 1), mlp2_w.T, mlp2_b.reshape(-1, 1))

    out = pl.pallas_call(
        _out_kernel,
        grid=(ngrid,),
        in_specs=[
            pl.BlockSpec((dm, _RC), lambda i: (0, i)),
            pl.BlockSpec((dm, dm), lambda i: (0, 0)),
            pl.BlockSpec((1, dm), lambda i: (0, 0)),
        ],
        out_specs=pl.BlockSpec((_RC, dm), lambda i: (i, 0)),
        out_shape=jax.ShapeDtypeStruct((rows, dm), f32),
        compiler_params=pltpu.CompilerParams(
            dimension_semantics=("parallel",),
        ),
    )(combined_t.reshape(dm, rows), out_w.astype(bf16), out_b.reshape(1, -1))

    return out.reshape(bsz, t, dm)


# speedup vs baseline: 6.8129x; 1.0186x over previous
from itertools import combinations

import numpy as np
import jax
import jax.numpy as jnp
from jax.experimental import pallas as pl
from jax.experimental.pallas import tpu as pltpu

_D = 1024
_H = 16
_DH = 64
_P = 4
_PD = 6
_NG = 21
_NGP = 24
_DECAY = 0.99
_C = 256
_KC = 256
_RC = 512
_G = 16
_SCALE = _DH ** -0.5
_LN_DECAY = float(np.log(_DECAY))
_DECAY_C = float(_DECAY ** _C)

_PAIRS = list(combinations(range(_P), 2))
_TI, _TJ = np.triu_indices(_PD)


def _np_consts():
    ea = np.zeros((8, 8), np.float32)
    eb = np.zeros((8, 8), np.float32)
    ec = np.zeros((8, 8), np.float32)
    ed = np.zeros((8, 8), np.float32)
    for kk, (i, j) in enumerate(_PAIRS):
        ea[kk, i] = 1.0
        eb[kk, 4 + j] = 1.0
        ec[kk, j] = 1.0
        ed[kk, 4 + i] = 1.0
    eti = np.zeros((_NGP, 8), np.float32)
    etj = np.zeros((_NGP, 8), np.float32)
    for kk in range(_NG):
        eti[kk, _TI[kk]] = 1.0
        etj[kk, _TJ[kk]] = 1.0
    i = np.arange(_C)[:, None]
    s = np.arange(_C)[None, :]
    lmat = np.where(s < i, _DECAY ** np.maximum(i - 1 - s, 0), 0.0).astype(np.float32)
    lt = np.ascontiguousarray(lmat.T)
    dvec = (_DECAY ** (_C - 1 - np.arange(_C))).astype(np.float32).reshape(1, _C)
    return ea, eb, ec, ed, eti, etj, lt, dvec


_EA, _EB, _EC, _ED, _ETI, _ETJ, _LT, _DVEC = _np_consts()


def _proj_kernel(x_ref, qw_ref, qb_ref, w1_ref, w2_ref, gw_ref, gb_ref,
                 qkv_ref, p1_ref, p2_ref, gl_ref):
    xb = x_ref[...]
    cdims = (((0,), (0,)), ((), ()))
    qkv = jax.lax.dot_general(qw_ref[...], xb, cdims,
                              preferred_element_type=jnp.float32) + qb_ref[...]
    qkv_ref[...] = qkv.astype(jnp.bfloat16)
    p1_ref[...] = jax.lax.dot_general(w1_ref[...], xb, cdims,
                                      preferred_element_type=jnp.float32)
    p2_ref[...] = jax.lax.dot_general(w2_ref[...], xb, cdims,
                                      preferred_element_type=jnp.float32)
    gl_ref[...] = jax.lax.dot_general(gw_ref[...], xb, cdims,
                                      preferred_element_type=jnp.float32) + gb_ref[...]


def _attn_gram_kernel(q_ref, k_ref, v_ref, pw_ref, gl_ref, lt_ref, dv_ref,
                      ea_ref, eb_ref, ec_ref, ed_ref, eti_ref, etj_ref,
                      m1_ref, m1b_ref, m2_ref, m2b_ref,
                      out_ref, s_ref):
    qc = pl.program_id(1)
    t0 = qc * _C
    f32 = jnp.float32
    bf16 = jnp.bfloat16

    @pl.when(qc == 0)
    def _():
        s_ref[...] = jnp.zeros((_G * _NGP, 1), f32)

    cdA = (((0,), (0,)), ((), ()))
    cdS = (((1,), (0,)), ((), ()))

    def one_head_chunk(g, off, m, l, acc, masked):
        kc = k_ref[g, :, pl.ds(off, _KC)]
        st = jax.lax.dot_general(kc, q_ref[g], cdA,
                                 preferred_element_type=f32) * _SCALE
        if masked:
            ki = jax.lax.broadcasted_iota(jnp.int32, (_KC, _C), 0)
            qi = jax.lax.broadcasted_iota(jnp.int32, (_KC, _C), 1)
            st = jnp.where(ki > qi, -1e30, st)
        m_new = jnp.maximum(m, jnp.max(st, axis=0, keepdims=True))
        alpha = jnp.exp(m - m_new)
        p = jnp.exp(st - m_new)
        l_new = l * alpha + jnp.sum(p, axis=0, keepdims=True)
        vc = v_ref[g, :, pl.ds(off, _KC)]
        acc_new = acc * alpha + jax.lax.dot_general(
            vc, p.astype(bf16), cdS, preferred_element_type=f32)
        return m_new, l_new, acc_new

    def body(j, carry):
        off = pl.multiple_of(j * _KC, _KC)
        return tuple(one_head_chunk(g, off, *carry[g], masked=False)
                     for g in range(_G))

    init = tuple((jnp.full((1, _C), -1e30, f32), jnp.zeros((1, _C), f32),
                  jnp.zeros((_DH, _C), f32)) for _ in range(_G))
    carry = jax.lax.fori_loop(0, qc, body, init)
    seqs = []
    for g in range(_G):
        m, l, acc = one_head_chunk(g, t0, *carry[g], masked=True)
        seqs.append(acc / l)

    dp = jnp.exp(jax.lax.broadcasted_iota(jnp.int32, (_NGP, _C), 1).astype(f32)
                 * _LN_DECAY)
    for g in range(_G):
        pw = pw_ref[g]
        a = jnp.dot(ea_ref[...], pw, preferred_element_type=f32)
        b = jnp.dot(eb_ref[...], pw, preferred_element_type=f32)
        c = jnp.dot(ec_ref[...], pw, preferred_element_type=f32)
        d = jnp.dot(ed_ref[...], pw, preferred_element_type=f32)
        parts = a * b - c * d
        s2 = jnp.sum(parts * parts, axis=0, keepdims=True)
        nr = jnp.maximum(jnp.sqrt(s2), 1e-12)
        wl = parts / nr
        u = jnp.dot(eti_ref[...], wl, preferred_element_type=f32)
        v = jnp.dot(etj_ref[...], wl, preferred_element_type=f32)
        o = u * v

        carry_s = s_ref[g * _NGP:(g + 1) * _NGP, :]
        gf = dp * carry_s + jnp.dot(o, lt_ref[...], preferred_element_type=f32)
        s_ref[g * _NGP:(g + 1) * _NGP, :] = (
            _DECAY_C * carry_s + jnp.sum(o * dv_ref[...], axis=1, keepdims=True))

        pre = jnp.dot(m1_ref[...], gf, preferred_element_type=f32) + m1b_ref[...]
        h1 = 0.5 * pre * (1.0 + jax.lax.erf(pre * 0.7071067811865476))
        mem = jnp.dot(m2_ref[...], h1, preferred_element_type=f32) + m2b_ref[...]

        gate = jax.nn.sigmoid(gl_ref[0, g:g + 1, :])
        out_ref[g] = (seqs[g] + gate * mem).astype(bf16)


def _out_kernel(c_ref, w_ref, b_ref, o_ref):
    o_ref[...] = jax.lax.dot_general(
        c_ref[...], w_ref[...], (((0,), (0,)), ((), ())),
        preferred_element_type=jnp.float32) + b_ref[...]


def kernel(x, qkv_w, qkv_b, w1_w, w2_w, mlp1_w, mlp1_b, mlp2_w, mlp2_b,
           gate_w, gate_b, out_w, out_b):
    bsz, t, dm = x.shape
    f32 = jnp.float32
    bf16 = jnp.bfloat16
    rows = bsz * t
    ngrid = rows // _RC
    nq = t // _C
    hh = _H

    xt = jnp.transpose(x.reshape(rows, dm).astype(bf16))

    qkvt, p1t, p2t, glt = pl.pallas_call(
        _proj_kernel,
        grid=(ngrid,),
        in_specs=[
            pl.BlockSpec((dm, _RC), lambda i: (0, i)),
            pl.BlockSpec((dm, 3 * dm), lambda i: (0, 0)),
            pl.BlockSpec((3 * dm, 1), lambda i: (0, 0)),
            pl.BlockSpec((dm, _H * _P), lambda i: (0, 0)),
            pl.BlockSpec((dm, _H * _P), lambda i: (0, 0)),
            pl.BlockSpec((dm, _H), lambda i: (0, 0)),
            pl.BlockSpec((_H, 1), lambda i: (0, 0)),
        ],
        out_specs=[
            pl.BlockSpec((3 * dm, _RC), lambda i: (0, i)),
            pl.BlockSpec((_H * _P, _RC), lambda i: (0, i)),
            pl.BlockSpec((_H * _P, _RC), lambda i: (0, i)),
            pl.BlockSpec((_H, _RC), lambda i: (0, i)),
        ],
        out_shape=[
            jax.ShapeDtypeStruct((3 * dm, rows), bf16),
            jax.ShapeDtypeStruct((_H * _P, rows), f32),
            jax.ShapeDtypeStruct((_H * _P, rows), f32),
            jax.ShapeDtypeStruct((_H, rows), f32),
        ],
        compiler_params=pltpu.CompilerParams(
            dimension_semantics=("parallel",),
        ),
    )(xt, qkv_w.astype(bf16), qkv_b.reshape(-1, 1), w1_w.astype(bf16),
      w2_w.astype(bf16), gate_w.astype(bf16), gate_b.reshape(-1, 1))

    qkvh = qkvt.reshape(3 * _H, _DH, rows)
    p1b = p1t.reshape(_H, _P, bsz, t)
    p1s = jnp.concatenate([jnp.zeros((_H, _P, bsz, 1), f32), p1b[..., :-1]], axis=3)
    p2b = p2t.reshape(_H, _P, bsz, t)
    pwt = jnp.concatenate([p1s, p2b], axis=1).reshape(_H, 8, rows)

    m1tp = jnp.concatenate([mlp1_w.T, jnp.zeros((_DH, _NGP - _NG), f32)], axis=1)

    hgn = _H // _G
    combined_t = pl.pallas_call(
        _attn_gram_kernel,
        grid=(bsz * hgn, nq),
        in_specs=[
            pl.BlockSpec((_G, _DH, _C), lambda bh, qc: (bh % hgn, 0, (bh // hgn) * nq + qc)),
            pl.BlockSpec((_G, _DH, t), lambda bh, qc: (hgn + bh % hgn, 0, bh // hgn)),
            pl.BlockSpec((_G, _DH, t), lambda bh, qc: (2 * hgn + bh % hgn, 0, bh // hgn)),
            pl.BlockSpec((_G, 8, _C), lambda bh, qc: (bh % hgn, 0, (bh // hgn) * nq + qc)),
            pl.BlockSpec((1, _G, _C), lambda bh, qc: (bh % hgn, 0, (bh // hgn) * nq + qc)),
            pl.BlockSpec((_C, _C), lambda bh, qc: (0, 0)),
            pl.BlockSpec((1, _C), lambda bh, qc: (0, 0)),
            pl.BlockSpec((8, 8), lambda bh, qc: (0, 0)),
            pl.BlockSpec((8, 8), lambda bh, qc: (0, 0)),
            pl.BlockSpec((8, 8), lambda bh, qc: (0, 0)),
            pl.BlockSpec((8, 8), lambda bh, qc: (0, 0)),
            pl.BlockSpec((_NGP, 8), lambda bh, qc: (0, 0)),
            pl.BlockSpec((_NGP, 8), lambda bh, qc: (0, 0)),
            pl.BlockSpec((_DH, _NGP), lambda bh, qc: (0, 0)),
            pl.BlockSpec((_DH, 1), lambda bh, qc: (0, 0)),
            pl.BlockSpec((_DH, _DH), lambda bh, qc: (0, 0)),
            pl.BlockSpec((_DH, 1), lambda bh, qc: (0, 0)),
        ],
        out_specs=pl.BlockSpec((_G, _DH, _C), lambda bh, qc: (bh % hgn, 0, (bh // hgn) * nq + qc)),
        out_shape=jax.ShapeDtypeStruct((_H, _DH, rows), bf16),
        scratch_shapes=[pltpu.VMEM((_G * _NGP, 1), f32)],
        compiler_params=pltpu.CompilerParams(
            dimension_semantics=("parallel", "arbitrary"),
        ),
    )(qkvh, qkvh, qkvh, pwt, glt.reshape(hgn, _G, rows),
      jnp.asarray(_LT), jnp.asarray(_DVEC), jnp.asarray(_EA), jnp.asarray(_EB),
      jnp.asarray(_EC), jnp.asarray(_ED), jnp.asarray(_ETI), jnp.asarray(_ETJ),
      m1tp.astype(f32), mlp1_b.reshape(-1, 1), mlp2_w.T, mlp2_b.reshape(-1, 1))

    out = pl.pallas_call(
        _out_kernel,
        grid=(ngrid,),
        in_specs=[
            pl.BlockSpec((dm, _RC), lambda i: (0, i)),
            pl.BlockSpec((dm, dm), lambda i: (0, 0)),
            pl.BlockSpec((1, dm), lambda i: (0, 0)),
        ],
        out_specs=pl.BlockSpec((_RC, dm), lambda i: (i, 0)),
        out_shape=jax.ShapeDtypeStruct((rows, dm), f32),
        compiler_params=pltpu.CompilerParams(
            dimension_semantics=("parallel",),
        ),
    )(combined_t.reshape(dm, rows), out_w.astype(bf16), out_b.reshape(1, -1))

    return out.reshape(bsz, t, dm)


# speedup vs baseline: 8.1252x; 1.1926x over previous
from itertools import combinations

import numpy as np
import jax
import jax.numpy as jnp
from jax.experimental import pallas as pl
from jax.experimental.pallas import tpu as pltpu

_D = 1024
_H = 16
_DH = 64
_P = 4
_PD = 6
_NG = 21
_NGP = 24
_DECAY = 0.99
_C = 256
_KC = 256
_RC = 512
_G = 16
_SCALE = _DH ** -0.5
_LN_DECAY = float(np.log(_DECAY))
_DECAY_C = float(_DECAY ** _C)

_PAIRS = list(combinations(range(_P), 2))
_TI, _TJ = np.triu_indices(_PD)


def _np_consts():
    ea = np.zeros((8, 8), np.float32)
    eb = np.zeros((8, 8), np.float32)
    ec = np.zeros((8, 8), np.float32)
    ed = np.zeros((8, 8), np.float32)
    for kk, (i, j) in enumerate(_PAIRS):
        ea[kk, i] = 1.0
        eb[kk, 4 + j] = 1.0
        ec[kk, j] = 1.0
        ed[kk, 4 + i] = 1.0
    eti = np.zeros((_NGP, 8), np.float32)
    etj = np.zeros((_NGP, 8), np.float32)
    for kk in range(_NG):
        eti[kk, _TI[kk]] = 1.0
        etj[kk, _TJ[kk]] = 1.0
    eye = np.eye(_G, dtype=np.float32)
    ea16 = np.kron(eye, ea)
    eb16 = np.kron(eye, eb)
    ec16 = np.kron(eye, ec)
    ed16 = np.kron(eye, ed)
    seg16 = np.kron(eye, np.ones((8, 8), np.float32))
    eti16 = np.kron(eye, eti)
    etj16 = np.kron(eye, etj)
    i = np.arange(_C)[:, None]
    s = np.arange(_C)[None, :]
    lmat = np.where(s < i, _DECAY ** np.maximum(i - 1 - s, 0), 0.0).astype(np.float32)
    lt = np.ascontiguousarray(lmat.T)
    dvec = (_DECAY ** (_C - 1 - np.arange(_C))).astype(np.float32).reshape(1, _C)
    return ea16, eb16, ec16, ed16, seg16, eti16, etj16, lt, dvec


_EA, _EB, _EC, _ED, _SEG, _ETI, _ETJ, _LT, _DVEC = _np_consts()


def _proj_kernel(x_ref, qw_ref, qb_ref, w1_ref, w2_ref, gw_ref, gb_ref,
                 qkv_ref, p1_ref, p2_ref, gl_ref):
    xb = x_ref[...]
    cdims = (((0,), (0,)), ((), ()))
    qkv = jax.lax.dot_general(qw_ref[...], xb, cdims,
                              preferred_element_type=jnp.float32) + qb_ref[...]
    qkv_ref[...] = qkv.astype(jnp.bfloat16)
    p1_ref[...] = jax.lax.dot_general(w1_ref[...], xb, cdims,
                                      preferred_element_type=jnp.float32)
    p2_ref[...] = jax.lax.dot_general(w2_ref[...], xb, cdims,
                                      preferred_element_type=jnp.float32)
    gl_ref[...] = jax.lax.dot_general(gw_ref[...], xb, cdims,
                                      preferred_element_type=jnp.float32) + gb_ref[...]


def _attn_gram_kernel(q_ref, k_ref, v_ref, pw_ref, gl_ref, lt_ref, dv_ref,
                      ea_ref, eb_ref, ec_ref, ed_ref, seg_ref, eti_ref, etj_ref,
                      m1_ref, m1b_ref, m2_ref, m2b_ref,
                      out_ref, s_ref):
    qc = pl.program_id(1)
    t0 = qc * _C
    f32 = jnp.float32
    bf16 = jnp.bfloat16

    @pl.when(qc == 0)
    def _():
        s_ref[...] = jnp.zeros((_G * _NGP, 1), f32)

    cdA = (((0,), (0,)), ((), ()))
    cdS = (((1,), (0,)), ((), ()))
    ones_row = jnp.ones((8, _KC), bf16)

    def one_head_chunk(g, off, m, acc, masked):
        kc = k_ref[g, :, pl.ds(off, _KC)]
        st = jax.lax.dot_general(kc, q_ref[g], cdA,
                                 preferred_element_type=f32) * _SCALE
        if masked:
            ki = jax.lax.broadcasted_iota(jnp.int32, (_KC, _C), 0)
            qi = jax.lax.broadcasted_iota(jnp.int32, (_KC, _C), 1)
            st = jnp.where(ki > qi, -1e30, st)
        m_new = jnp.maximum(m, jnp.max(st, axis=0, keepdims=True))
        alpha = jnp.exp(m - m_new)
        p = jnp.exp(st - m_new)
        va = jnp.concatenate([v_ref[g, :, pl.ds(off, _KC)], ones_row], axis=0)
        acc_new = acc * alpha + jax.lax.dot_general(
            va, p.astype(bf16), cdS, preferred_element_type=f32)
        return m_new, acc_new

    def body(j, carry):
        off = pl.multiple_of(j * _KC, _KC)
        return tuple(one_head_chunk(g, off, *carry[g], masked=False)
                     for g in range(_G))

    init = tuple((jnp.full((1, _C), -1e30, f32),
                  jnp.zeros((_DH + 8, _C), f32)) for _ in range(_G))
    carry = jax.lax.fori_loop(0, qc, body, init)
    seqs = []
    for g in range(_G):
        m, acc = one_head_chunk(g, t0, *carry[g], masked=True)
        seqs.append(acc[0:_DH] / acc[_DH:_DH + 1])

    pwa = pw_ref[...]
    a = jnp.dot(ea_ref[...], pwa, preferred_element_type=f32)
    b = jnp.dot(eb_ref[...], pwa, preferred_element_type=f32)
    c = jnp.dot(ec_ref[...], pwa, preferred_element_type=f32)
    d = jnp.dot(ed_ref[...], pwa, preferred_element_type=f32)
    parts = a * b - c * d
    s2 = jnp.dot(seg_ref[...], (parts * parts).astype(bf16),
                 preferred_element_type=f32)
    nr = jnp.maximum(jnp.sqrt(s2), 1e-12)
    wl = (parts / nr).astype(bf16)
    u = jnp.dot(eti_ref[...], wl, preferred_element_type=f32)
    v = jnp.dot(etj_ref[...], wl, preferred_element_type=f32)
    o = u * v

    carry_s = s_ref[...]
    dp = jnp.exp(jax.lax.broadcasted_iota(jnp.int32, (_G * _NGP, _C), 1).astype(f32)
                 * _LN_DECAY)
    gf = dp * carry_s + jnp.dot(o.astype(bf16), lt_ref[...],
                                preferred_element_type=f32)
    s_ref[...] = _DECAY_C * carry_s + jnp.sum(o * dv_ref[...], axis=1, keepdims=True)

    pre = jnp.dot(m1_ref[...], gf.astype(bf16),
                  preferred_element_type=f32) + m1b_ref[...]
    h1 = 0.5 * pre * (1.0 + jax.lax.erf(pre * 0.7071067811865476))
    mem = jnp.dot(m2_ref[...], h1.astype(bf16),
                  preferred_element_type=f32) + m2b_ref[...]

    for g in range(_G):
        gate = jax.nn.sigmoid(gl_ref[0, g:g + 1, :])
        out_ref[g] = (seqs[g] + gate * mem[g * _DH:(g + 1) * _DH]).astype(bf16)


def _out_kernel(c_ref, w_ref, b_ref, o_ref):
    o_ref[...] = jax.lax.dot_general(
        c_ref[...], w_ref[...], (((0,), (0,)), ((), ())),
        preferred_element_type=jnp.float32) + b_ref[...]


def kernel(x, qkv_w, qkv_b, w1_w, w2_w, mlp1_w, mlp1_b, mlp2_w, mlp2_b,
           gate_w, gate_b, out_w, out_b):
    bsz, t, dm = x.shape
    f32 = jnp.float32
    bf16 = jnp.bfloat16
    rows = bsz * t
    ngrid = rows // _RC
    nq = t // _C
    hh = _H

    xt = jnp.transpose(x.reshape(rows, dm).astype(bf16))

    qkvt, p1t, p2t, glt = pl.pallas_call(
        _proj_kernel,
        grid=(ngrid,),
        in_specs=[
            pl.BlockSpec((dm, _RC), lambda i: (0, i)),
            pl.BlockSpec((dm, 3 * dm), lambda i: (0, 0)),
            pl.BlockSpec((3 * dm, 1), lambda i: (0, 0)),
            pl.BlockSpec((dm, _H * _P), lambda i: (0, 0)),
            pl.BlockSpec((dm, _H * _P), lambda i: (0, 0)),
            pl.BlockSpec((dm, _H), lambda i: (0, 0)),
            pl.BlockSpec((_H, 1), lambda i: (0, 0)),
        ],
        out_specs=[
            pl.BlockSpec((3 * dm, _RC), lambda i: (0, i)),
            pl.BlockSpec((_H * _P, _RC), lambda i: (0, i)),
            pl.BlockSpec((_H * _P, _RC), lambda i: (0, i)),
            pl.BlockSpec((_H, _RC), lambda i: (0, i)),
        ],
        out_shape=[
            jax.ShapeDtypeStruct((3 * dm, rows), bf16),
            jax.ShapeDtypeStruct((_H * _P, rows), f32),
            jax.ShapeDtypeStruct((_H * _P, rows), f32),
            jax.ShapeDtypeStruct((_H, rows), f32),
        ],
        compiler_params=pltpu.CompilerParams(
            dimension_semantics=("parallel",),
        ),
    )(xt, qkv_w.astype(bf16), qkv_b.reshape(-1, 1), w1_w.astype(bf16),
      w2_w.astype(bf16), gate_w.astype(bf16), gate_b.reshape(-1, 1))

    qkvh = qkvt.reshape(3 * _H, _DH, rows)
    p1b = p1t.reshape(_H, _P, bsz, t)
    p1s = jnp.concatenate([jnp.zeros((_H, _P, bsz, 1), f32), p1b[..., :-1]], axis=3)
    p2b = p2t.reshape(_H, _P, bsz, t)
    pwt = jnp.concatenate([p1s, p2b], axis=1).reshape(_H, 8, rows)

    m1tp = jnp.concatenate([mlp1_w.T, jnp.zeros((_DH, _NGP - _NG), f32)], axis=1)

    hgn = _H // _G
    combined_t = pl.pallas_call(
        _attn_gram_kernel,
        grid=(bsz * hgn, nq),
        in_specs=[
            pl.BlockSpec((_G, _DH, _C), lambda bh, qc: (bh % hgn, 0, (bh // hgn) * nq + qc)),
            pl.BlockSpec((_G, _DH, t), lambda bh, qc: (hgn + bh % hgn, 0, bh // hgn)),
            pl.BlockSpec((_G, _DH, t), lambda bh, qc: (2 * hgn + bh % hgn, 0, bh // hgn)),
            pl.BlockSpec((_G * 8, _C), lambda bh, qc: (bh % hgn, (bh // hgn) * nq + qc)),
            pl.BlockSpec((1, _G, _C), lambda bh, qc: (bh % hgn, 0, (bh // hgn) * nq + qc)),
            pl.BlockSpec((_C, _C), lambda bh, qc: (0, 0)),
            pl.BlockSpec((1, _C), lambda bh, qc: (0, 0)),
            pl.BlockSpec((_G * 8, _G * 8), lambda bh, qc: (0, 0)),
            pl.BlockSpec((_G * 8, _G * 8), lambda bh, qc: (0, 0)),
            pl.BlockSpec((_G * 8, _G * 8), lambda bh, qc: (0, 0)),
            pl.BlockSpec((_G * 8, _G * 8), lambda bh, qc: (0, 0)),
            pl.BlockSpec((_G * 8, _G * 8), lambda bh, qc: (0, 0)),
            pl.BlockSpec((_G * _NGP, _G * 8), lambda bh, qc: (0, 0)),
            pl.BlockSpec((_G * _NGP, _G * 8), lambda bh, qc: (0, 0)),
            pl.BlockSpec((_G * _DH, _G * _NGP), lambda bh, qc: (0, 0)),
            pl.BlockSpec((_G * _DH, 1), lambda bh, qc: (0, 0)),
            pl.BlockSpec((_G * _DH, _G * _DH), lambda bh, qc: (0, 0)),
            pl.BlockSpec((_G * _DH, 1), lambda bh, qc: (0, 0)),
        ],
        out_specs=pl.BlockSpec((_G, _DH, _C), lambda bh, qc: (bh % hgn, 0, (bh // hgn) * nq + qc)),
        out_shape=jax.ShapeDtypeStruct((_H, _DH, rows), bf16),
        scratch_shapes=[pltpu.VMEM((_G * _NGP, 1), f32)],
        compiler_params=pltpu.CompilerParams(
            dimension_semantics=("parallel", "arbitrary"),
        ),
    )(qkvh, qkvh, qkvh, pwt.reshape(_H * 8, rows).astype(bf16),
      glt.reshape(hgn, _G, rows),
      jnp.asarray(_LT).astype(bf16), jnp.asarray(_DVEC),
      jnp.asarray(_EA).astype(bf16), jnp.asarray(_EB).astype(bf16),
      jnp.asarray(_EC).astype(bf16), jnp.asarray(_ED).astype(bf16),
      jnp.asarray(_SEG).astype(bf16),
      jnp.asarray(_ETI).astype(bf16), jnp.asarray(_ETJ).astype(bf16),
      jnp.kron(jnp.eye(_G, dtype=f32), m1tp).astype(bf16),
      jnp.tile(mlp1_b.reshape(-1, 1), (_G, 1)),
      jnp.kron(jnp.eye(_G, dtype=f32), mlp2_w.T).astype(bf16),
      jnp.tile(mlp2_b.reshape(-1, 1), (_G, 1)))

    out = pl.pallas_call(
        _out_kernel,
        grid=(ngrid,),
        in_specs=[
            pl.BlockSpec((dm, _RC), lambda i: (0, i)),
            pl.BlockSpec((dm, dm), lambda i: (0, 0)),
            pl.BlockSpec((1, dm), lambda i: (0, 0)),
        ],
        out_specs=pl.BlockSpec((_RC, dm), lambda i: (i, 0)),
        out_shape=jax.ShapeDtypeStruct((rows, dm), f32),
        compiler_params=pltpu.CompilerParams(
            dimension_semantics=("parallel",),
        ),
    )(combined_t.reshape(dm, rows), out_w.astype(bf16), out_b.reshape(1, -1))

    return out.reshape(bsz, t, dm)


# speedup vs baseline: 10.8793x; 1.3390x over previous
from itertools import combinations

import numpy as np
import jax
import jax.numpy as jnp
from jax.experimental import pallas as pl
from jax.experimental.pallas import tpu as pltpu

_D = 1024
_H = 16
_DH = 64
_P = 4
_PD = 6
_NG = 21
_NGP = 24
_DECAY = 0.99
_C = 512
_KC = 256
_RC = 512
_G = 16
_SCALE = _DH ** -0.5
_LN_DECAY = float(np.log(_DECAY))
_DECAY_C = float(_DECAY ** _C)

_PAIRS = list(combinations(range(_P), 2))
_TI, _TJ = np.triu_indices(_PD)


def _np_consts():
    ea = np.zeros((8, 8), np.float32)
    eb = np.zeros((8, 8), np.float32)
    ec = np.zeros((8, 8), np.float32)
    ed = np.zeros((8, 8), np.float32)
    for kk, (i, j) in enumerate(_PAIRS):
        ea[kk, i] = 1.0
        eb[kk, 4 + j] = 1.0
        ec[kk, j] = 1.0
        ed[kk, 4 + i] = 1.0
    eti = np.zeros((_NGP, 8), np.float32)
    etj = np.zeros((_NGP, 8), np.float32)
    for kk in range(_NG):
        eti[kk, _TI[kk]] = 1.0
        etj[kk, _TJ[kk]] = 1.0
    eye = np.eye(_G, dtype=np.float32)
    ea16 = np.kron(eye, ea)
    eb16 = np.kron(eye, eb)
    ec16 = np.kron(eye, ec)
    ed16 = np.kron(eye, ed)
    seg16 = np.kron(eye, np.ones((8, 8), np.float32))
    eti16 = np.kron(eye, eti)
    etj16 = np.kron(eye, etj)
    i = np.arange(_C)[:, None]
    s = np.arange(_C)[None, :]
    lmat = np.where(s < i, _DECAY ** np.maximum(i - 1 - s, 0), 0.0).astype(np.float32)
    lt = np.ascontiguousarray(lmat.T)
    dvec = (_DECAY ** (_C - 1 - np.arange(_C))).astype(np.float32).reshape(1, _C)
    return ea16, eb16, ec16, ed16, seg16, eti16, etj16, lt, dvec


_EA, _EB, _EC, _ED, _SEG, _ETI, _ETJ, _LT, _DVEC = _np_consts()


def _proj_kernel(x_ref, qw_ref, qb_ref, w1_ref, w2_ref, gw_ref, gb_ref,
                 qkv_ref, p1_ref, p2_ref, gl_ref):
    xb = x_ref[...]
    cdims = (((0,), (0,)), ((), ()))
    qkv = jax.lax.dot_general(qw_ref[...], xb, cdims,
                              preferred_element_type=jnp.float32) + qb_ref[...]
    qkv_ref[...] = qkv.astype(jnp.bfloat16)
    p1_ref[...] = jax.lax.dot_general(w1_ref[...], xb, cdims,
                                      preferred_element_type=jnp.float32)
    p2_ref[...] = jax.lax.dot_general(w2_ref[...], xb, cdims,
                                      preferred_element_type=jnp.float32)
    gl_ref[...] = jax.lax.dot_general(gw_ref[...], xb, cdims,
                                      preferred_element_type=jnp.float32) + gb_ref[...]


def _attn_gram_kernel(q_ref, k_ref, v_ref, pw_ref, gl_ref, lt_ref, dv_ref,
                      ea_ref, eb_ref, ec_ref, ed_ref, seg_ref, eti_ref, etj_ref,
                      m1_ref, m1b_ref, m2_ref, m2b_ref,
                      out_ref, s_ref):
    qc = pl.program_id(1)
    t0 = qc * _C
    f32 = jnp.float32
    bf16 = jnp.bfloat16

    @pl.when(qc == 0)
    def _():
        s_ref[...] = jnp.zeros((_G * _NGP, 1), f32)

    cdA = (((0,), (0,)), ((), ()))
    cdS = (((1,), (0,)), ((), ()))
    ones_row = jnp.ones((8, _KC), bf16)

    def one_head_chunk(g, off, m, acc, masked, moff=0):
        kc = k_ref[g, :, pl.ds(off, _KC)]
        st = jax.lax.dot_general(kc, q_ref[g], cdA,
                                 preferred_element_type=f32) * _SCALE
        if masked:
            ki = jax.lax.broadcasted_iota(jnp.int32, (_KC, _C), 0)
            qi = jax.lax.broadcasted_iota(jnp.int32, (_KC, _C), 1)
            st = jnp.where(ki + moff > qi, -1e30, st)
        m_new = jnp.maximum(m, jnp.max(st, axis=0, keepdims=True))
        alpha = jnp.exp(m - m_new)
        p = jnp.exp(st - m_new)
        va = jnp.concatenate([v_ref[g, :, pl.ds(off, _KC)], ones_row], axis=0)
        acc_new = acc * alpha + jax.lax.dot_general(
            va, p.astype(bf16), cdS, preferred_element_type=f32)
        return m_new, acc_new

    def body(j, carry):
        off = pl.multiple_of(j * _KC, _KC)
        return tuple(one_head_chunk(g, off, *carry[g], masked=False)
                     for g in range(_G))

    init = tuple((jnp.full((1, _C), -1e30, f32),
                  jnp.zeros((_DH + 8, _C), f32)) for _ in range(_G))
    carry = jax.lax.fori_loop(0, (_C // _KC) * qc, body, init)
    seqs = []
    for g in range(_G):
        m, acc = carry[g]
        for dj in range(_C // _KC):
            m, acc = one_head_chunk(g, t0 + dj * _KC, m, acc,
                                    masked=True, moff=dj * _KC)
        seqs.append(acc[0:_DH] / acc[_DH:_DH + 1])

    pwa = pw_ref[...]
    a = jnp.dot(ea_ref[...], pwa, preferred_element_type=f32)
    b = jnp.dot(eb_ref[...], pwa, preferred_element_type=f32)
    c = jnp.dot(ec_ref[...], pwa, preferred_element_type=f32)
    d = jnp.dot(ed_ref[...], pwa, preferred_element_type=f32)
    parts = a * b - c * d
    s2 = jnp.dot(seg_ref[...], (parts * parts).astype(bf16),
                 preferred_element_type=f32)
    nr = jnp.maximum(jnp.sqrt(s2), 1e-12)
    wl = (parts / nr).astype(bf16)
    u = jnp.dot(eti_ref[...], wl, preferred_element_type=f32)
    v = jnp.dot(etj_ref[...], wl, preferred_element_type=f32)
    o = u * v

    carry_s = s_ref[...]
    dp = jnp.exp(jax.lax.broadcasted_iota(jnp.int32, (_G * _NGP, _C), 1).astype(f32)
                 * _LN_DECAY)
    gf = dp * carry_s + jnp.dot(o.astype(bf16), lt_ref[...],
                                preferred_element_type=f32)
    s_ref[...] = _DECAY_C * carry_s + jnp.sum(o * dv_ref[...], axis=1, keepdims=True)

    pre = jnp.dot(m1_ref[...], gf.astype(bf16),
                  preferred_element_type=f32) + m1b_ref[...]
    h1 = 0.5 * pre * (1.0 + jax.lax.erf(pre * 0.7071067811865476))
    mem = jnp.dot(m2_ref[...], h1.astype(bf16),
                  preferred_element_type=f32) + m2b_ref[...]

    for g in range(_G):
        gate = jax.nn.sigmoid(gl_ref[0, g:g + 1, :])
        out_ref[g] = (seqs[g] + gate * mem[g * _DH:(g + 1) * _DH]).astype(bf16)


def _out_kernel(c_ref, w_ref, b_ref, o_ref):
    o_ref[...] = jax.lax.dot_general(
        c_ref[...], w_ref[...], (((0,), (0,)), ((), ())),
        preferred_element_type=jnp.float32) + b_ref[...]


def kernel(x, qkv_w, qkv_b, w1_w, w2_w, mlp1_w, mlp1_b, mlp2_w, mlp2_b,
           gate_w, gate_b, out_w, out_b):
    bsz, t, dm = x.shape
    f32 = jnp.float32
    bf16 = jnp.bfloat16
    rows = bsz * t
    ngrid = rows // _RC
    nq = t // _C
    hh = _H

    xt = jnp.transpose(x.reshape(rows, dm).astype(bf16))

    qkvt, p1t, p2t, glt = pl.pallas_call(
        _proj_kernel,
        grid=(ngrid,),
        in_specs=[
            pl.BlockSpec((dm, _RC), lambda i: (0, i)),
            pl.BlockSpec((dm, 3 * dm), lambda i: (0, 0)),
            pl.BlockSpec((3 * dm, 1), lambda i: (0, 0)),
            pl.BlockSpec((dm, _H * _P), lambda i: (0, 0)),
            pl.BlockSpec((dm, _H * _P), lambda i: (0, 0)),
            pl.BlockSpec((dm, _H), lambda i: (0, 0)),
            pl.BlockSpec((_H, 1), lambda i: (0, 0)),
        ],
        out_specs=[
            pl.BlockSpec((3 * dm, _RC), lambda i: (0, i)),
            pl.BlockSpec((_H * _P, _RC), lambda i: (0, i)),
            pl.BlockSpec((_H * _P, _RC), lambda i: (0, i)),
            pl.BlockSpec((_H, _RC), lambda i: (0, i)),
        ],
        out_shape=[
            jax.ShapeDtypeStruct((3 * dm, rows), bf16),
            jax.ShapeDtypeStruct((_H * _P, rows), f32),
            jax.ShapeDtypeStruct((_H * _P, rows), f32),
            jax.ShapeDtypeStruct((_H, rows), f32),
        ],
        compiler_params=pltpu.CompilerParams(
            dimension_semantics=("parallel",),
        ),
    )(xt, qkv_w.astype(bf16), qkv_b.reshape(-1, 1), w1_w.astype(bf16),
      w2_w.astype(bf16), gate_w.astype(bf16), gate_b.reshape(-1, 1))

    qkvh = qkvt.reshape(3 * _H, _DH, rows)
    p1b = p1t.reshape(_H, _P, bsz, t)
    p1s = jnp.concatenate([jnp.zeros((_H, _P, bsz, 1), f32), p1b[..., :-1]], axis=3)
    p2b = p2t.reshape(_H, _P, bsz, t)
    pwt = jnp.concatenate([p1s, p2b], axis=1).reshape(_H, 8, rows)

    m1tp = jnp.concatenate([mlp1_w.T, jnp.zeros((_DH, _NGP - _NG), f32)], axis=1)

    hgn = _H // _G
    combined_t = pl.pallas_call(
        _attn_gram_kernel,
        grid=(bsz * hgn, nq),
        in_specs=[
            pl.BlockSpec((_G, _DH, _C), lambda bh, qc: (bh % hgn, 0, (bh // hgn) * nq + qc)),
            pl.BlockSpec((_G, _DH, t), lambda bh, qc: (hgn + bh % hgn, 0, bh // hgn)),
            pl.BlockSpec((_G, _DH, t), lambda bh, qc: (2 * hgn + bh % hgn, 0, bh // hgn)),
            pl.BlockSpec((_G * 8, _C), lambda bh, qc: (bh % hgn, (bh // hgn) * nq + qc)),
            pl.BlockSpec((1, _G, _C), lambda bh, qc: (bh % hgn, 0, (bh // hgn) * nq + qc)),
            pl.BlockSpec((_C, _C), lambda bh, qc: (0, 0)),
            pl.BlockSpec((1, _C), lambda bh, qc: (0, 0)),
            pl.BlockSpec((_G * 8, _G * 8), lambda bh, qc: (0, 0)),
            pl.BlockSpec((_G * 8, _G * 8), lambda bh, qc: (0, 0)),
            pl.BlockSpec((_G * 8, _G * 8), lambda bh, qc: (0, 0)),
            pl.BlockSpec((_G * 8, _G * 8), lambda bh, qc: (0, 0)),
            pl.BlockSpec((_G * 8, _G * 8), lambda bh, qc: (0, 0)),
            pl.BlockSpec((_G * _NGP, _G * 8), lambda bh, qc: (0, 0)),
            pl.BlockSpec((_G * _NGP, _G * 8), lambda bh, qc: (0, 0)),
            pl.BlockSpec((_G * _DH, _G * _NGP), lambda bh, qc: (0, 0)),
            pl.BlockSpec((_G * _DH, 1), lambda bh, qc: (0, 0)),
            pl.BlockSpec((_G * _DH, _G * _DH), lambda bh, qc: (0, 0)),
            pl.BlockSpec((_G * _DH, 1), lambda bh, qc: (0, 0)),
        ],
        out_specs=pl.BlockSpec((_G, _DH, _C), lambda bh, qc: (bh % hgn, 0, (bh // hgn) * nq + qc)),
        out_shape=jax.ShapeDtypeStruct((_H, _DH, rows), bf16),
        scratch_shapes=[pltpu.VMEM((_G * _NGP, 1), f32)],
        compiler_params=pltpu.CompilerParams(
            dimension_semantics=("parallel", "arbitrary"),
        ),
    )(qkvh, qkvh, qkvh, pwt.reshape(_H * 8, rows).astype(bf16),
      glt.reshape(hgn, _G, rows),
      jnp.asarray(_LT).astype(bf16), jnp.asarray(_DVEC),
      jnp.asarray(_EA).astype(bf16), jnp.asarray(_EB).astype(bf16),
      jnp.asarray(_EC).astype(bf16), jnp.asarray(_ED).astype(bf16),
      jnp.asarray(_SEG).astype(bf16),
      jnp.asarray(_ETI).astype(bf16), jnp.asarray(_ETJ).astype(bf16),
      jnp.kron(jnp.eye(_G, dtype=f32), m1tp).astype(bf16),
      jnp.tile(mlp1_b.reshape(-1, 1), (_G, 1)),
      jnp.kron(jnp.eye(_G, dtype=f32), mlp2_w.T).astype(bf16),
      jnp.tile(mlp2_b.reshape(-1, 1), (_G, 1)))

    out = pl.pallas_call(
        _out_kernel,
        grid=(ngrid,),
        in_specs=[
            pl.BlockSpec((dm, _RC), lambda i: (0, i)),
            pl.BlockSpec((dm, dm), lambda i: (0, 0)),
            pl.BlockSpec((1, dm), lambda i: (0, 0)),
        ],
        out_specs=pl.BlockSpec((_RC, dm), lambda i: (i, 0)),
        out_shape=jax.ShapeDtypeStruct((rows, dm), f32),
        compiler_params=pltpu.CompilerParams(
            dimension_semantics=("parallel",),
        ),
    )(combined_t.reshape(dm, rows), out_w.astype(bf16), out_b.reshape(1, -1))

    return out.reshape(bsz, t, dm)


# speedup vs baseline: 13.0198x; 1.1967x over previous
from itertools import combinations

import numpy as np
import jax
import jax.numpy as jnp
from jax.experimental import pallas as pl
from jax.experimental.pallas import tpu as pltpu

_D = 1024
_H = 16
_DH = 64
_P = 4
_PD = 6
_NG = 21
_NGP = 24
_DECAY = 0.99
_C = 512
_KC = 512
_RC = 512
_G = 16
_SCALE = _DH ** -0.5
_LN_DECAY = float(np.log(_DECAY))
_DECAY_C = float(_DECAY ** _C)

_PAIRS = list(combinations(range(_P), 2))
_TI, _TJ = np.triu_indices(_PD)


def _np_consts():
    ea = np.zeros((8, 8), np.float32)
    eb = np.zeros((8, 8), np.float32)
    ec = np.zeros((8, 8), np.float32)
    ed = np.zeros((8, 8), np.float32)
    for kk, (i, j) in enumerate(_PAIRS):
        ea[kk, i] = 1.0
        eb[kk, 4 + j] = 1.0
        ec[kk, j] = 1.0
        ed[kk, 4 + i] = 1.0
    eti = np.zeros((_NGP, 8), np.float32)
    etj = np.zeros((_NGP, 8), np.float32)
    for kk in range(_NG):
        eti[kk, _TI[kk]] = 1.0
        etj[kk, _TJ[kk]] = 1.0
    eye = np.eye(_G, dtype=np.float32)
    ea16 = np.kron(eye, ea)
    eb16 = np.kron(eye, eb)
    ec16 = np.kron(eye, ec)
    ed16 = np.kron(eye, ed)
    seg16 = np.kron(eye, np.ones((8, 8), np.float32))
    eti16 = np.kron(eye, eti)
    etj16 = np.kron(eye, etj)
    i = np.arange(_C)[:, None]
    s = np.arange(_C)[None, :]
    lmat = np.where(s < i, _DECAY ** np.maximum(i - 1 - s, 0), 0.0).astype(np.float32)
    lt = np.ascontiguousarray(lmat.T)
    dvec = (_DECAY ** (_C - 1 - np.arange(_C))).astype(np.float32).reshape(1, _C)
    return ea16, eb16, ec16, ed16, seg16, eti16, etj16, lt, dvec


_EA, _EB, _EC, _ED, _SEG, _ETI, _ETJ, _LT, _DVEC = _np_consts()


def _proj_kernel(x_ref, qw_ref, qb_ref, w1_ref, w2_ref, gw_ref, gb_ref,
                 qkv_ref, p1_ref, p2_ref, gl_ref):
    xb = x_ref[...]
    cdims = (((0,), (0,)), ((), ()))
    qkv = jax.lax.dot_general(qw_ref[...], xb, cdims,
                              preferred_element_type=jnp.float32) + qb_ref[...]
    qkv_ref[...] = qkv.astype(jnp.bfloat16)
    p1_ref[...] = jax.lax.dot_general(w1_ref[...], xb, cdims,
                                      preferred_element_type=jnp.float32)
    p2_ref[...] = jax.lax.dot_general(w2_ref[...], xb, cdims,
                                      preferred_element_type=jnp.float32)
    gl_ref[...] = jax.lax.dot_general(gw_ref[...], xb, cdims,
                                      preferred_element_type=jnp.float32) + gb_ref[...]


def _attn_gram_kernel(q_ref, k_ref, v_ref, pw_ref, gl_ref, lt_ref, dv_ref,
                      ea_ref, eb_ref, ec_ref, ed_ref, seg_ref, eti_ref, etj_ref,
                      m1_ref, m1b_ref, m2_ref, m2b_ref,
                      out_ref, s_ref):
    qc = pl.program_id(1)
    t0 = qc * _C
    f32 = jnp.float32
    bf16 = jnp.bfloat16

    @pl.when(qc == 0)
    def _():
        s_ref[...] = jnp.zeros((_G * _NGP, 1), f32)

    cdA = (((0,), (0,)), ((), ()))
    cdS = (((1,), (0,)), ((), ()))
    ones_row = jnp.ones((8, _KC), bf16)

    def one_head_chunk(g, off, m, acc, masked, moff=0):
        kc = k_ref[g, :, pl.ds(off, _KC)]
        st = jax.lax.dot_general(kc, q_ref[g], cdA,
                                 preferred_element_type=f32) * _SCALE
        if masked:
            ki = jax.lax.broadcasted_iota(jnp.int32, (_KC, _C), 0)
            qi = jax.lax.broadcasted_iota(jnp.int32, (_KC, _C), 1)
            st = jnp.where(ki + moff > qi, -1e30, st)
        m_new = jnp.maximum(m, jnp.max(st, axis=0, keepdims=True))
        alpha = jnp.exp(m - m_new)
        p = jnp.exp(st - m_new)
        va = jnp.concatenate([v_ref[g, :, pl.ds(off, _KC)], ones_row], axis=0)
        acc_new = acc * alpha + jax.lax.dot_general(
            va, p.astype(bf16), cdS, preferred_element_type=f32)
        return m_new, acc_new

    def body(j, carry):
        off = pl.multiple_of(j * _KC, _KC)
        return tuple(one_head_chunk(g, off, *carry[g], masked=False)
                     for g in range(_G))

    init = tuple((jnp.full((1, _C), -1e30, f32),
                  jnp.zeros((_DH + 8, _C), f32)) for _ in range(_G))
    carry = jax.lax.fori_loop(0, (_C // _KC) * qc, body, init)
    seqs = []
    for g in range(_G):
        m, acc = carry[g]
        for dj in range(_C // _KC):
            m, acc = one_head_chunk(g, t0 + dj * _KC, m, acc,
                                    masked=True, moff=dj * _KC)
        seqs.append(acc[0:_DH] / acc[_DH:_DH + 1])

    pwa = pw_ref[...]
    a = jnp.dot(ea_ref[...], pwa, preferred_element_type=f32)
    b = jnp.dot(eb_ref[...], pwa, preferred_element_type=f32)
    c = jnp.dot(ec_ref[...], pwa, preferred_element_type=f32)
    d = jnp.dot(ed_ref[...], pwa, preferred_element_type=f32)
    parts = a * b - c * d
    s2 = jnp.dot(seg_ref[...], (parts * parts).astype(bf16),
                 preferred_element_type=f32)
    nr = jnp.maximum(jnp.sqrt(s2), 1e-12)
    wl = (parts / nr).astype(bf16)
    u = jnp.dot(eti_ref[...], wl, preferred_element_type=f32)
    v = jnp.dot(etj_ref[...], wl, preferred_element_type=f32)
    o = u * v

    carry_s = s_ref[...]
    dp = jnp.exp(jax.lax.broadcasted_iota(jnp.int32, (_G * _NGP, _C), 1).astype(f32)
                 * _LN_DECAY)
    gf = dp * carry_s + jnp.dot(o.astype(bf16), lt_ref[...],
                                preferred_element_type=f32)
    s_ref[...] = _DECAY_C * carry_s + jnp.sum(o * dv_ref[...], axis=1, keepdims=True)

    pre = jnp.dot(m1_ref[...], gf.astype(bf16),
                  preferred_element_type=f32) + m1b_ref[...]
    h1 = 0.5 * pre * (1.0 + jax.lax.erf(pre * 0.7071067811865476))
    mem = jnp.dot(m2_ref[...], h1.astype(bf16),
                  preferred_element_type=f32) + m2b_ref[...]

    for g in range(_G):
        gate = jax.nn.sigmoid(gl_ref[0, g:g + 1, :])
        out_ref[g] = (seqs[g] + gate * mem[g * _DH:(g + 1) * _DH]).astype(bf16)


def _out_kernel(c_ref, w_ref, b_ref, o_ref):
    o_ref[...] = jax.lax.dot_general(
        c_ref[...], w_ref[...], (((0,), (0,)), ((), ())),
        preferred_element_type=jnp.float32) + b_ref[...]


def kernel(x, qkv_w, qkv_b, w1_w, w2_w, mlp1_w, mlp1_b, mlp2_w, mlp2_b,
           gate_w, gate_b, out_w, out_b):
    bsz, t, dm = x.shape
    f32 = jnp.float32
    bf16 = jnp.bfloat16
    rows = bsz * t
    ngrid = rows // _RC
    nq = t // _C
    hh = _H

    xt = jnp.transpose(x.reshape(rows, dm).astype(bf16))

    qkvt, p1t, p2t, glt = pl.pallas_call(
        _proj_kernel,
        grid=(ngrid,),
        in_specs=[
            pl.BlockSpec((dm, _RC), lambda i: (0, i)),
            pl.BlockSpec((dm, 3 * dm), lambda i: (0, 0)),
            pl.BlockSpec((3 * dm, 1), lambda i: (0, 0)),
            pl.BlockSpec((dm, _H * _P), lambda i: (0, 0)),
            pl.BlockSpec((dm, _H * _P), lambda i: (0, 0)),
            pl.BlockSpec((dm, _H), lambda i: (0, 0)),
            pl.BlockSpec((_H, 1), lambda i: (0, 0)),
        ],
        out_specs=[
            pl.BlockSpec((3 * dm, _RC), lambda i: (0, i)),
            pl.BlockSpec((_H * _P, _RC), lambda i: (0, i)),
            pl.BlockSpec((_H * _P, _RC), lambda i: (0, i)),
            pl.BlockSpec((_H, _RC), lambda i: (0, i)),
        ],
        out_shape=[
            jax.ShapeDtypeStruct((3 * dm, rows), bf16),
            jax.ShapeDtypeStruct((_H * _P, rows), f32),
            jax.ShapeDtypeStruct((_H * _P, rows), f32),
            jax.ShapeDtypeStruct((_H, rows), f32),
        ],
        compiler_params=pltpu.CompilerParams(
            dimension_semantics=("parallel",),
        ),
    )(xt, qkv_w.astype(bf16), qkv_b.reshape(-1, 1), w1_w.astype(bf16),
      w2_w.astype(bf16), gate_w.astype(bf16), gate_b.reshape(-1, 1))

    qkvh = qkvt.reshape(3 * _H, _DH, rows)
    p1b = p1t.reshape(_H, _P, bsz, t)
    p1s = jnp.concatenate([jnp.zeros((_H, _P, bsz, 1), f32), p1b[..., :-1]], axis=3)
    p2b = p2t.reshape(_H, _P, bsz, t)
    pwt = jnp.concatenate([p1s, p2b], axis=1).reshape(_H, 8, rows)

    m1tp = jnp.concatenate([mlp1_w.T, jnp.zeros((_DH, _NGP - _NG), f32)], axis=1)

    hgn = _H // _G
    combined_t = pl.pallas_call(
        _attn_gram_kernel,
        grid=(bsz * hgn, nq),
        in_specs=[
            pl.BlockSpec((_G, _DH, _C), lambda bh, qc: (bh % hgn, 0, (bh // hgn) * nq + qc)),
            pl.BlockSpec((_G, _DH, t), lambda bh, qc: (hgn + bh % hgn, 0, bh // hgn)),
            pl.BlockSpec((_G, _DH, t), lambda bh, qc: (2 * hgn + bh % hgn, 0, bh // hgn)),
            pl.BlockSpec((_G * 8, _C), lambda bh, qc: (bh % hgn, (bh // hgn) * nq + qc)),
            pl.BlockSpec((1, _G, _C), lambda bh, qc: (bh % hgn, 0, (bh // hgn) * nq + qc)),
            pl.BlockSpec((_C, _C), lambda bh, qc: (0, 0)),
            pl.BlockSpec((1, _C), lambda bh, qc: (0, 0)),
            pl.BlockSpec((_G * 8, _G * 8), lambda bh, qc: (0, 0)),
            pl.BlockSpec((_G * 8, _G * 8), lambda bh, qc: (0, 0)),
            pl.BlockSpec((_G * 8, _G * 8), lambda bh, qc: (0, 0)),
            pl.BlockSpec((_G * 8, _G * 8), lambda bh, qc: (0, 0)),
            pl.BlockSpec((_G * 8, _G * 8), lambda bh, qc: (0, 0)),
            pl.BlockSpec((_G * _NGP, _G * 8), lambda bh, qc: (0, 0)),
            pl.BlockSpec((_G * _NGP, _G * 8), lambda bh, qc: (0, 0)),
            pl.BlockSpec((_G * _DH, _G * _NGP), lambda bh, qc: (0, 0)),
            pl.BlockSpec((_G * _DH, 1), lambda bh, qc: (0, 0)),
            pl.BlockSpec((_G * _DH, _G * _DH), lambda bh, qc: (0, 0)),
            pl.BlockSpec((_G * _DH, 1), lambda bh, qc: (0, 0)),
        ],
        out_specs=pl.BlockSpec((_G, _DH, _C), lambda bh, qc: (bh % hgn, 0, (bh // hgn) * nq + qc)),
        out_shape=jax.ShapeDtypeStruct((_H, _DH, rows), bf16),
        scratch_shapes=[pltpu.VMEM((_G * _NGP, 1), f32)],
        compiler_params=pltpu.CompilerParams(
            dimension_semantics=("parallel", "arbitrary"),
        ),
    )(qkvh, qkvh, qkvh, pwt.reshape(_H * 8, rows).astype(bf16),
      glt.reshape(hgn, _G, rows),
      jnp.asarray(_LT).astype(bf16), jnp.asarray(_DVEC),
      jnp.asarray(_EA).astype(bf16), jnp.asarray(_EB).astype(bf16),
      jnp.asarray(_EC).astype(bf16), jnp.asarray(_ED).astype(bf16),
      jnp.asarray(_SEG).astype(bf16),
      jnp.asarray(_ETI).astype(bf16), jnp.asarray(_ETJ).astype(bf16),
      jnp.kron(jnp.eye(_G, dtype=f32), m1tp).astype(bf16),
      jnp.tile(mlp1_b.reshape(-1, 1), (_G, 1)),
      jnp.kron(jnp.eye(_G, dtype=f32), mlp2_w.T).astype(bf16),
      jnp.tile(mlp2_b.reshape(-1, 1), (_G, 1)))

    out = pl.pallas_call(
        _out_kernel,
        grid=(ngrid,),
        in_specs=[
            pl.BlockSpec((dm, _RC), lambda i: (0, i)),
            pl.BlockSpec((dm, dm), lambda i: (0, 0)),
            pl.BlockSpec((1, dm), lambda i: (0, 0)),
        ],
        out_specs=pl.BlockSpec((_RC, dm), lambda i: (i, 0)),
        out_shape=jax.ShapeDtypeStruct((rows, dm), f32),
        compiler_params=pltpu.CompilerParams(
            dimension_semantics=("parallel",),
        ),
    )(combined_t.reshape(dm, rows), out_w.astype(bf16), out_b.reshape(1, -1))

    return out.reshape(bsz, t, dm)
